# baseline (speedup 1.0000x reference)
import jax
import jax.numpy as jnp
from jax.experimental import pallas as pl

EMB = 128
K_LOCAL = 50


def _linear(x, W, b):
    return x @ W + b


def _rel_feat_embed(feat, agents, unobserved, W, b):
    pos = jnp.concatenate([feat[..., :2] - agents[:, :, None, :2], feat[..., 2:4]], axis=-1)
    x = pos * (1.0 - unobserved)
    return _linear(x, W, b), pos


def _rel_embed(feat, agents, W, b):
    B, N = feat.shape[0], feat.shape[1]
    A = agents.shape[1]
    dxy = feat[:, None, :, :2] - agents[:, :, None, :2]
    hd = jnp.broadcast_to(feat[:, None, :, 2:4], (B, A, N, 2))
    pos = jnp.concatenate([dxy, hd], axis=-1)
    return _linear(pos, W, b), pos


def kernel(map_features, map_masks, light_features, light_masks, stop_features, stop_masks,
           walker_features, walker_masks, agents_features, agents_masks,
           W_map, b_map, W_m2a, b_m2a, W_light, b_light, W_l2a, b_l2a,
           W_stop, b_stop, W_s2a, b_s2a, W_walk, b_walk, W_w2a, b_w2a,
           W_ego, b_ego, W_ag, b_ag, W_a2a, b_a2a):
    B, A = agents_masks.shape
    dist = jnp.linalg.norm(map_features[:, None, :, :2] - agents_features[:, :, None, :2], axis=-1)
    dist = jnp.where(map_masks[:, None, :], jnp.inf, dist)
    _, closest = jax.lax.top_k(-dist, K_LOCAL)
    bidx = jnp.arange(B)[:, None, None]
    m2a_feat = map_features[bidx, closest]
    m2a_masks = map_masks[bidx, closest]
    ego_d = dist[:, 0][bidx, closest]
    unobserved = (ego_d > 1.0)[..., None].astype(jnp.float32)
    m2a_emb, m2a_pos = _rel_feat_embed(m2a_feat, agents_features, unobserved, W_m2a, b_m2a)
    m2a_emb = m2a_emb + _linear(m2a_feat[..., :7], W_map, b_map)
    m2a_masks = m2a_masks | (jnp.linalg.norm(m2a_pos[..., :2], axis=-1) > 1.0)
    lights_emb = _linear(light_features, W_light, b_light)
    l2a_emb, l2a_pos = _rel_embed(light_features, agents_features, W_l2a, b_l2a)
    l2a_emb = l2a_emb + lights_emb[:, None]
    l2a_masks = light_masks[:, None] | (jnp.linalg.norm(l2a_pos[..., :2], axis=-1) > 1.0)
    stops_emb = _linear(stop_features, W_stop, b_stop)
    s2a_emb, s2a_pos = _rel_embed(stop_features, agents_features, W_s2a, b_s2a)
    s2a_emb = s2a_emb + stops_emb[:, None]
    s2a_masks = stop_masks[:, None] | (jnp.linalg.norm(s2a_pos[..., :2], axis=-1) > 1.0)
    walkers_emb = _linear(walker_features, W_walk, b_walk)
    w2a_emb, w2a_pos = _rel_embed(walker_features, agents_features, W_w2a, b_w2a)
    w2a_emb = w2a_emb + walkers_emb[:, None]
    w2a_masks = walker_masks[:, None] | (jnp.linalg.norm(w2a_pos[..., :2], axis=-1) > 1.0)
    scene_emb = jnp.concatenate([m2a_emb, l2a_emb, s2a_emb, w2a_emb], axis=2)
    scene_masks = jnp.concatenate([m2a_masks, l2a_masks, s2a_masks, w2a_masks], axis=2)
    ego_emb = _linear(agents_features[:, :1], W_ego, b_ego)
    other_emb = _linear(agents_features[:, 1:], W_ag, b_ag)
    agents_emb = jnp.concatenate([ego_emb, other_emb], axis=1)
    a2a_emb, a2a_pos = _rel_embed(agents_features, agents_features, W_a2a, b_a2a)
    a2a_masks = agents_masks[:, None] | (jnp.linalg.norm(a2a_pos[..., :2], axis=-1) > 1.0)
    return scene_emb, scene_masks, agents_emb, a2a_emb, a2a_masks


# speedup vs baseline: 3.3091x; 3.3091x over previous
import functools

import jax
import jax.numpy as jnp
from jax import lax
from jax.experimental import pallas as pl
from jax.experimental.pallas import tpu as pltpu
from jax.experimental.pallas import tpu_sc as plsc

EMB = 128
K_LOCAL = 50
B, M, A = 16, 20000, 64
CL = 128
NC = 160
MP = NC * CL
CAP = 128
AB = 8
NQ = B * A
INF = float("inf")


NVP = NC + 16
NSEL = 96


def _k1_body(mapt_ref, maskt_ref, ag_ref, d2_ref, vmin_ref):
    qx = ag_ref[0, :, 0:1]
    qy = ag_ref[0, :, 1:2]
    mx = mapt_ref[0, 0:1, :]
    my = mapt_ref[0, 1:2, :]
    mk = maskt_ref[0, 0:1, :]
    dx = mx - qx
    dy = my - qy
    d2 = dx * dx + dy * dy
    d2 = jnp.where(mk > 0.5, INF, d2)
    d2_ref[0] = d2
    vm = jnp.min(d2.reshape(AB, NC, CL), axis=-1)

    def step(_, t):
        masked = jnp.where(vm > t, vm, INF)
        return jnp.min(masked, axis=-1, keepdims=True)

    t = lax.fori_loop(0, K_LOCAL, step, jnp.full((AB, 1), -INF, jnp.float32))
    vmin_ref[0, :, 0:NC] = vm
    vmin_ref[0, :, NC:NVP] = jnp.broadcast_to(t, (AB, 16))


def _run_k1(mapt, maskt, agents):
    return pl.pallas_call(
        _k1_body,
        grid=(B, A // AB),
        in_specs=[
            pl.BlockSpec((1, 2, MP), lambda b, j: (b, 0, 0)),
            pl.BlockSpec((1, 1, MP), lambda b, j: (b, 0, 0)),
            pl.BlockSpec((1, AB, 7), lambda b, j: (b, j, 0)),
        ],
        out_specs=[
            pl.BlockSpec((1, AB, MP), lambda b, j: (b, j, 0)),
            pl.BlockSpec((1, AB, NVP), lambda b, j: (b, j, 0)),
        ],
        out_shape=[
            jax.ShapeDtypeStruct((B, A, MP), jnp.float32),
            jax.ShapeDtypeStruct((B, A, NVP), jnp.float32),
        ],
    )(mapt, maskt, agents)


def _gather16(x, idx):
    return x.at[idx].get(mode="promise_in_bounds")


def _compact16(vals, msk, lane, pad):
    rank = jnp.where(msk, 1, 0).astype(jnp.int32)
    for k in (1, 2, 4, 8):
        sh = _gather16(rank, jnp.maximum(lane - k, 0))
        rank = rank + jnp.where(lane >= k, sh, 0)
    tgt = lane + 1
    p = jnp.where(rank[7] < tgt, 8, 0).astype(jnp.int32)
    for step in (4, 2, 1):
        r = _gather16(rank, p + (step - 1))
        p = p + jnp.where(r < tgt, step, 0)
    n = rank[15]
    okm = tgt <= n
    out0 = jnp.where(okm, _gather16(vals[0], p), pad)
    outs = [out0] + [_gather16(v, p) for v in vals[1:]]
    return outs, n


def _k2_body(vmin_hbm, d2t_hbm, itab_hbm, cd_hbm, ci_hbm,
             vrow0, vrow1, idsl_v, idsg_v, gd2_v, gidx_v, cd_v, ci_v,
             sem0, sem1, semg, semo):
    info = plsc.get_sparse_core_info()
    nc_sc = info.num_cores
    wid = lax.axis_index("s") * nc_sc + lax.axis_index("c")
    qpw = NQ // (nc_sc * 16)
    qbase = wid * qpw
    lane = lax.iota(jnp.int32, 16)
    l15 = jnp.full((16,), 15, jnp.int32)
    vrows = (vrow0, vrow1)
    sems = (sem0, sem1)
    cp_cur = pltpu.async_copy(vmin_hbm.at[qbase], vrows[0], sems[0])

    for q in range(qpw):
        cp_next = None
        if q + 1 < qpw:
            cp_next = pltpu.async_copy(vmin_hbm.at[qbase + q + 1],
                                       vrows[(q + 1) % 2], sems[(q + 1) % 2])
        cp_cur.wait()
        vrow = vrows[q % 2]
        tau16 = vrow[pl.ds(NC, 16)]

        for j in range(NSEL // 16):
            idsl_v[pl.ds(j * 16, 16)] = jnp.full((16,), NC - 1, jnp.int32)

        def stepA(j, off):
            vm = vrow[pl.ds(j * 16, 16)]
            msk = vm <= tau16
            (ids,), n = _compact16([lane + j * 16], msk, lane, NC - 1)
            offc = jnp.minimum(off, NSEL)
            idsl_v[pl.ds(offc, 16)] = ids
            return off + n

        lax.fori_loop(0, NC // 16, stepA, jnp.int32(0))

        gbase = (qbase + q) * NC
        for j in range(NSEL // 16):
            idsg_v[pl.ds(j * 16, 16)] = idsl_v[pl.ds(j * 16, 16)] + gbase
        gc0 = pltpu.async_copy(d2t_hbm.at[idsg_v], gd2_v, semg)
        gc1 = pltpu.async_copy(itab_hbm.at[idsl_v.at[pl.ds(0, NSEL)]],
                               gidx_v, semg)
        gc0.wait()
        gc1.wait()
        for j in range(CAP // 16 + 1):
            cd_v[pl.ds(j * 16, 16)] = jnp.full((16,), INF, jnp.float32)
            ci_v[pl.ds(j * 16, 16)] = jnp.zeros((16,), jnp.int32)

        def stepB(j, off):
            row = gd2_v[j // 8, pl.ds((j % 8) * 16, 16)]
            idxv = gidx_v[j // 8, pl.ds((j % 8) * 16, 16)]
            msk = row <= tau16
            (crow, cidx), n = _compact16([row, idxv], msk, lane, INF)
            offc = jnp.minimum(off, CAP)
            cd_v[pl.ds(offc, 16)] = crow
            ci_v[pl.ds(offc, 16)] = cidx
            return off + n

        lax.fori_loop(0, NSEL * 8, stepB, jnp.int32(0))
        oc0 = pltpu.async_copy(cd_v.at[pl.ds(0, CAP)], cd_hbm.at[qbase + q],
                               semo)
        oc1 = pltpu.async_copy(ci_v.at[pl.ds(0, CAP)], ci_hbm.at[qbase + q],
                               semo)
        oc0.wait()
        oc1.wait()
        cp_cur = cp_next


def _run_k2(vmin_rows, d2t, itab):
    mesh = plsc.VectorSubcoreMesh(core_axis_name="c", subcore_axis_name="s")
    kern = functools.partial(
        pl.kernel,
        mesh=mesh,
        out_type=[
            jax.ShapeDtypeStruct((NQ, CAP), jnp.float32),
            jax.ShapeDtypeStruct((NQ, CAP), jnp.int32),
        ],
        scratch_types=[
            pltpu.VMEM((NVP,), jnp.float32),
            pltpu.VMEM((NVP,), jnp.float32),
            pltpu.VMEM((128,), jnp.int32),
            pltpu.VMEM((NSEL,), jnp.int32),
            pltpu.VMEM((NSEL, CL), jnp.float32),
            pltpu.VMEM((NSEL, CL), jnp.int32),
            pltpu.VMEM((CAP + 128,), jnp.float32),
            pltpu.VMEM((CAP + 128,), jnp.int32),
            pltpu.SemaphoreType.DMA,
            pltpu.SemaphoreType.DMA,
            pltpu.SemaphoreType.DMA,
            pltpu.SemaphoreType.DMA,
        ],
    )(_k2_body)
    return kern(vmin_rows, d2t, itab)


def _k3_body(cd_ref, ci_ref, gid_ref):
    b = pl.program_id(0)
    d2 = cd_ref[0]
    idx = ci_ref[0].astype(jnp.float32)
    valid = d2 < jnp.float32(1e38)
    dist = jnp.sqrt(d2)
    rank = jnp.zeros((AB, CAP), jnp.int32)
    for j in range(CAP):
        dj = dist[:, j:j + 1]
        ij = idx[:, j:j + 1]
        vj = valid[:, j:j + 1]
        less = jnp.logical_and(
            vj, jnp.logical_or(dj < dist,
                               jnp.logical_and(dj == dist, ij < idx)))
        rank = rank + jnp.where(less, 1, 0)
    rank = jnp.where(valid, rank, jnp.int32(10000))
    base = (b // (A // AB)) * M
    gid = idx.astype(jnp.int32) + base
    cols = []
    for k in range(K_LOCAL):
        sel = rank == k
        cols.append(jnp.sum(jnp.where(sel, gid, 0), axis=-1, keepdims=True))
    gid_ref[0] = jnp.concatenate(cols, axis=-1)


def _run_k3(cd, ci):
    return pl.pallas_call(
        _k3_body,
        grid=(NQ // AB,),
        in_specs=[
            pl.BlockSpec((1, AB, CAP), lambda i: (i, 0, 0)),
            pl.BlockSpec((1, AB, CAP), lambda i: (i, 0, 0)),
        ],
        out_specs=pl.BlockSpec((1, AB, K_LOCAL), lambda i: (i, 0, 0)),
        out_shape=jax.ShapeDtypeStruct((NQ // AB, AB, K_LOCAL), jnp.int32),
    )(cd, ci)


def _k4_body(tab_hbm, gid_hbm, out_hbm, idx_v, rows_v, sem, semo):
    info = plsc.get_sparse_core_info()
    nc_sc = info.num_cores
    wid = lax.axis_index("s") * nc_sc + lax.axis_index("c")
    rpw = (NQ * K_LOCAL) // (nc_sc * 16)
    base = wid * rpw
    pltpu.sync_copy(gid_hbm.at[pl.ds(base, rpw)], idx_v)
    copies = []
    nfull = rpw // 128
    for c in range(nfull):
        copies.append(pltpu.async_copy(
            tab_hbm.at[idx_v.at[pl.ds(c * 128, 128)]],
            rows_v.at[pl.ds(c * 128, 128), :], sem))
    rem = rpw - nfull * 128
    if rem:
        copies.append(pltpu.async_copy(
            tab_hbm.at[idx_v.at[pl.ds(nfull * 128, rem)]],
            rows_v.at[pl.ds(nfull * 128, rem), :], sem))
    for cp in copies:
        cp.wait()
    oc = pltpu.async_copy(rows_v, out_hbm.at[pl.ds(base, rpw), :], semo)
    oc.wait()


def _run_k4(tab, gid_flat):
    mesh = plsc.VectorSubcoreMesh(core_axis_name="c", subcore_axis_name="s")
    rpw = (NQ * K_LOCAL) // 32
    kern = functools.partial(
        pl.kernel,
        mesh=mesh,
        compiler_params=pltpu.CompilerParams(use_tc_tiling_on_sc=False),
        out_type=jax.ShapeDtypeStruct((NQ * K_LOCAL, 8), jnp.float32),
        scratch_types=[
            pltpu.VMEM((rpw,), jnp.int32),
            pltpu.VMEM((rpw, 8), jnp.float32),
            pltpu.SemaphoreType.DMA,
            pltpu.SemaphoreType.DMA,
        ],
    )(_k4_body)
    return kern(tab, gid_flat)


def _mm(x3, W_ref, b_ref, kdim):
    acc = jnp.broadcast_to(b_ref[0:1, :][None], x3.shape[:2] + (EMB,))
    for k in range(kdim):
        acc = acc + x3[:, :, k:k + 1] * W_ref[k:k + 1, :][None]
    return acc


def _rel_part(feat, nfeat, axs, ays, mref, Wp_ref, bp_ref, We_ref, be_ref,
              kemb):
    N = feat.shape[0]
    fx = feat[:, 0:1][None]
    fy = feat[:, 1:2][None]
    px = fx - axs
    py = fy - ays
    pos = jnp.concatenate(
        [px, py,
         jnp.broadcast_to(feat[:, 2:3][None], (AB, N, 1)),
         jnp.broadcast_to(feat[:, 3:4][None], (AB, N, 1))], axis=-1)
    emb = _mm(pos, Wp_ref, bp_ref, 4)
    base = _mm(feat[None, :, :kemb], We_ref, be_ref, kemb)
    emb = emb + base
    dist = jnp.sqrt(px * px + py * py)
    msk = jnp.logical_or(mref[0, 0:1, :][:, :, None] > 0.5,
                         dist > 1.0)[:, :, 0]
    return emb, msk.astype(jnp.float32)


def _k5_body(m2a_ref, agt_ref, agf_ref, li_ref, st_ref, wa_ref,
             lm_ref, sm_ref, wm_ref, am_ref,
             wmap_ref, bmap_ref, wm2a_ref, bm2a_ref,
             wlight_ref, blight_ref, wl2a_ref, bl2a_ref,
             wstop_ref, bstop_ref, ws2a_ref, bs2a_ref,
             wwalk_ref, bwalk_ref, ww2a_ref, bw2a_ref,
             wego_ref, bego_ref, wag_ref, bag_ref, wa2a_ref, ba2a_ref,
             m2a_o, l2a_o, s2a_o, w2a_o, a2a_o, aemb_o,
             mm_o, lm_o, sm_o, wm_o, am_o):
    j = pl.program_id(1)
    agt = agt_ref[0]
    agf = agf_ref[0]
    axs = agt[:, 0:1][:, None, :]
    ays = agt[:, 1:2][:, None, :]

    f3 = m2a_ref[0].reshape(AB, K_LOCAL, 8)
    px = f3[:, :, 0:1] - axs
    py = f3[:, :, 1:2] - ays
    e0x = f3[:, :, 0:1] - agf[0, 0]
    e0y = f3[:, :, 1:2] - agf[0, 1]
    ego_d = jnp.sqrt(e0x * e0x + e0y * e0y)
    gmask = f3[:, :, 7:8]
    unobs = jnp.logical_or(ego_d > 1.0, gmask > 0.5)
    obs = jnp.where(unobs, 0.0, 1.0)
    pos4 = jnp.concatenate([px, py, f3[:, :, 2:3], f3[:, :, 3:4]], axis=-1)
    m2a = _mm(pos4 * obs, wm2a_ref, bm2a_ref, 4) + _mm(
        f3[:, :, :7], wmap_ref, bmap_ref, 7)
    mdist = jnp.sqrt(px * px + py * py)
    m2a_o[0] = m2a
    mm_o[0] = jnp.logical_or(gmask > 0.5,
                             mdist > 1.0)[:, :, 0].astype(jnp.float32)

    l2a, lmsk = _rel_part(li_ref[0], 8, axs, ays, lm_ref, wl2a_ref, bl2a_ref,
                          wlight_ref, blight_ref, 8)
    l2a_o[0] = l2a
    lm_o[0] = lmsk
    s2a, smsk = _rel_part(st_ref[0], 5, axs, ays, sm_ref, ws2a_ref, bs2a_ref,
                          wstop_ref, bstop_ref, 5)
    s2a_o[0] = s2a
    sm_o[0] = smsk
    w2a, wmsk = _rel_part(wa_ref[0], 6, axs, ays, wm_ref, ww2a_ref, bw2a_ref,
                          wwalk_ref, bwalk_ref, 6)
    w2a_o[0] = w2a
    wm_o[0] = wmsk

    gx = agf[:, 0:1][None]
    gy = agf[:, 1:2][None]
    apx = gx - axs
    apy = gy - ays
    apos = jnp.concatenate(
        [apx, apy,
         jnp.broadcast_to(agf[:, 2:3][None], (AB, A, 1)),
         jnp.broadcast_to(agf[:, 3:4][None], (AB, A, 1))], axis=-1)
    a2a_o[0] = _mm(apos, wa2a_ref, ba2a_ref, 4)
    adist = jnp.sqrt(apx * apx + apy * apy)
    am_o[0] = jnp.logical_or(am_ref[0, 0:1, :][:, :, None] > 0.5,
                             adist > 1.0)[:, :, 0].astype(jnp.float32)

    ego = _mm(agt[None], wego_ref, bego_ref, 7)[0]
    oth = _mm(agt[None], wag_ref, bag_ref, 7)[0]
    rows = lax.broadcasted_iota(jnp.int32, (AB, 1), 0) + j * AB
    aemb_o[0] = jnp.where(rows == 0, ego, oth)


def _run_k5(m2a_rows, agents, lights, stops, walkers,
            lmask_f, smask_f, wmask_f, amask_f, weights):
    L, S, Wn = 16, 32, 64

    def wspec(w):
        return pl.BlockSpec(w.shape, lambda b, j: (0,) * w.ndim)

    win = []
    wargs = []
    for w in weights:
        w2 = w.reshape(1, EMB) if w.ndim == 1 else w
        wargs.append(w2)
        win.append(wspec(w2))

    def bo(shape):
        nd = len(shape)
        return pl.BlockSpec((1, 1) + shape,
                            lambda b, j: (b, j) + (0,) * nd)

    outs = [
        (K_LOCAL, EMB), (L, EMB), (S, EMB), (Wn, EMB), (A, EMB), (EMB,),
        (K_LOCAL,), (L,), (S,), (Wn,), (A,),
    ]
    return pl.pallas_call(
        _k5_body,
        grid=(B, A // AB),
        in_specs=[
            pl.BlockSpec((1, AB * K_LOCAL, 8),
                         lambda b, j: (b * (A // AB) + j, 0, 0)),
            pl.BlockSpec((1, AB, 7), lambda b, j: (b, j, 0)),
            pl.BlockSpec((1, A, 7), lambda b, j: (b, 0, 0)),
            pl.BlockSpec((1, L, 8), lambda b, j: (b, 0, 0)),
            pl.BlockSpec((1, S, 5), lambda b, j: (b, 0, 0)),
            pl.BlockSpec((1, Wn, 6), lambda b, j: (b, 0, 0)),
            pl.BlockSpec((1, 1, L), lambda b, j: (b, 0, 0)),
            pl.BlockSpec((1, 1, S), lambda b, j: (b, 0, 0)),
            pl.BlockSpec((1, 1, Wn), lambda b, j: (b, 0, 0)),
            pl.BlockSpec((1, 1, A), lambda b, j: (b, 0, 0)),
        ] + win,
        out_specs=[
            pl.BlockSpec((1, AB) + sh, (lambda b, j, n=len(sh): (b, j) + (0,) * n))
            for sh in outs
        ],
        out_shape=[
            jax.ShapeDtypeStruct((B, A) + sh, jnp.float32) for sh in outs
        ],
    )(m2a_rows.reshape(NQ // AB, AB * K_LOCAL, 8), agents, agents,
      lights, stops, walkers, lmask_f, smask_f, wmask_f, amask_f, *wargs)


def kernel(map_features, map_masks, light_features, light_masks, stop_features,
           stop_masks, walker_features, walker_masks, agents_features,
           agents_masks, W_map, b_map, W_m2a, b_m2a, W_light, b_light, W_l2a,
           b_l2a, W_stop, b_stop, W_s2a, b_s2a, W_walk, b_walk, W_w2a, b_w2a,
           W_ego, b_ego, W_ag, b_ag, W_a2a, b_a2a):
    mapt = jnp.transpose(map_features[:, :, :2], (0, 2, 1))
    mapt = jnp.pad(mapt, ((0, 0), (0, 0), (0, MP - M)))
    maskf = map_masks.astype(jnp.float32)[:, None, :]
    maskt = jnp.pad(maskf, ((0, 0), (0, 0), (0, MP - M)),
                    constant_values=1.0)
    tab = jnp.concatenate(
        [map_features, map_masks.astype(jnp.float32)[:, :, None]],
        axis=-1).reshape(B * M, 8)

    d2, vmin = _run_k1(mapt, maskt, agents_features)
    d2t = d2.reshape(NQ * NC, CL)
    vmin_rows = vmin.reshape(NQ, NVP)

    itab = jnp.arange(NC * CL, dtype=jnp.int32).reshape(NC, CL)
    cd, ci = _run_k2(vmin_rows, d2t, itab)
    cd3 = cd.reshape(NQ // AB, AB, CAP)
    ci3 = ci.reshape(NQ // AB, AB, CAP)

    gid = _run_k3(cd3, ci3)
    gid_flat = gid.reshape(NQ * K_LOCAL)

    rows = _run_k4(tab, gid_flat)
    m2a_rows = rows

    weights = (W_map, b_map, W_m2a, b_m2a, W_light, b_light, W_l2a, b_l2a,
               W_stop, b_stop, W_s2a, b_s2a, W_walk, b_walk, W_w2a, b_w2a,
               W_ego, b_ego, W_ag, b_ag, W_a2a, b_a2a)
    (m2a_e, l2a_e, s2a_e, w2a_e, a2a_emb, agents_emb,
     mm_f, lm_f, sm_f, wm_f, am_f) = _run_k5(
        m2a_rows, agents_features, light_features, stop_features,
        walker_features,
        light_masks.astype(jnp.float32)[:, None, :],
        stop_masks.astype(jnp.float32)[:, None, :],
        walker_masks.astype(jnp.float32)[:, None, :],
        agents_masks.astype(jnp.float32)[:, None, :],
        weights)
    scene_emb = jnp.concatenate([m2a_e, l2a_e, s2a_e, w2a_e], axis=2)
    scene_masks = jnp.concatenate([mm_f, lm_f, sm_f, wm_f], axis=2) > 0.5
    return (scene_emb, scene_masks, agents_emb, a2a_emb, am_f > 0.5)


# speedup vs baseline: 4.3662x; 1.3195x over previous
import functools

import jax
import jax.numpy as jnp
from jax import lax
from jax.experimental import pallas as pl
from jax.experimental.pallas import tpu as pltpu
from jax.experimental.pallas import tpu_sc as plsc

EMB = 128
K_LOCAL = 50
B, M, A = 16, 20000, 64
CL = 128
NC = 160
MP = NC * CL
CAP = 128
AB = 8
NQ = B * A
INF = float("inf")


NVP = NC + 16
NSEL = 64


def _k1_body(mapt_ref, maskt_ref, ag_ref, d2_ref, vmin_ref):
    qx = ag_ref[0, :, 0:1]
    qy = ag_ref[0, :, 1:2]
    mx = mapt_ref[0, 0:1, :]
    my = mapt_ref[0, 1:2, :]
    mk = maskt_ref[0, 0:1, :]
    dx = mx - qx
    dy = my - qy
    d2 = dx * dx + dy * dy
    d2 = jnp.where(mk > 0.5, INF, d2)
    d2_ref[0] = d2
    vm = jnp.min(d2.reshape(AB, NC, CL), axis=-1)

    def step(_, t):
        masked = jnp.where(vm > t, vm, INF)
        return jnp.min(masked, axis=-1, keepdims=True)

    t = lax.fori_loop(0, K_LOCAL, step, jnp.full((AB, 1), -INF, jnp.float32))
    vmin_ref[0, :, 0:NC] = vm
    vmin_ref[0, :, NC:NVP] = jnp.broadcast_to(t, (AB, 16))


def _run_k1(mapt, maskt, agents):
    return pl.pallas_call(
        _k1_body,
        grid=(B, A // AB),
        in_specs=[
            pl.BlockSpec((1, 2, MP), lambda b, j: (b, 0, 0)),
            pl.BlockSpec((1, 1, MP), lambda b, j: (b, 0, 0)),
            pl.BlockSpec((1, AB, 7), lambda b, j: (b, j, 0)),
        ],
        out_specs=[
            pl.BlockSpec((1, AB, MP), lambda b, j: (b, j, 0)),
            pl.BlockSpec((1, AB, NVP), lambda b, j: (b, j, 0)),
        ],
        out_shape=[
            jax.ShapeDtypeStruct((B, A, MP), jnp.float32),
            jax.ShapeDtypeStruct((B, A, NVP), jnp.float32),
        ],
    )(mapt, maskt, agents)


def _gather16(x, idx):
    return x.at[idx].get(mode="promise_in_bounds")


def _prefix16(msk, lane):
    rank = jnp.where(msk, 1, 0).astype(jnp.int32)
    for k in (1, 2, 4, 8):
        sh = _gather16(rank, jnp.maximum(lane - k, 0))
        rank = rank + jnp.where(lane >= k, sh, 0)
    return rank


def _compact16(vals, rank, lane, pad):
    tgt = lane + 1
    p = jnp.where(rank[7] < tgt, 8, 0).astype(jnp.int32)
    for step in (4, 2, 1):
        r = _gather16(rank, p + (step - 1))
        p = p + jnp.where(r < tgt, step, 0)
    okm = tgt <= rank[15]
    out0 = jnp.where(okm, _gather16(vals[0], p), pad)
    return [out0] + [_gather16(v, p) for v in vals[1:]]


def _k2_body(vmin_hbm, d2t_hbm, itab_hbm, cd_hbm, ci_hbm,
             vrow0, vrow1, idsl_v, idsg_v, gd2_v, gidx_v, cd_v, ci_v,
             sem0, sem1, semg, semo):
    info = plsc.get_sparse_core_info()
    nc_sc = info.num_cores
    wid = lax.axis_index("s") * nc_sc + lax.axis_index("c")
    qpw = NQ // (nc_sc * 16)
    qbase = wid * qpw
    lane = lax.iota(jnp.int32, 16)
    l15 = jnp.full((16,), 15, jnp.int32)
    vrows = (vrow0, vrow1)
    sems = (sem0, sem1)
    cp_cur = pltpu.async_copy(vmin_hbm.at[qbase], vrows[0], sems[0])

    for q in range(qpw):
        cp_next = None
        if q + 1 < qpw:
            cp_next = pltpu.async_copy(vmin_hbm.at[qbase + q + 1],
                                       vrows[(q + 1) % 2], sems[(q + 1) % 2])
        cp_cur.wait()
        vrow = vrows[q % 2]
        tau16 = vrow[pl.ds(NC, 16)]

        for j in range(NSEL // 16):
            idsl_v[pl.ds(j * 16, 16)] = jnp.full((16,), NC - 1, jnp.int32)

        def stepA(j, off):
            vm = vrow[pl.ds(j * 16, 16)]
            msk = vm <= tau16
            rank = _prefix16(msk, lane)
            n = rank[15]

            @pl.when(n > 0)
            def _():
                (ids,) = _compact16([lane + j * 16], rank, lane, NC - 1)
                offc = jnp.minimum(off, NSEL)
                idsl_v[pl.ds(offc, 16)] = ids

            return off + n

        na = lax.fori_loop(0, NC // 16, stepA, jnp.int32(0))
        na = jnp.minimum(na, NSEL)

        gbase = (qbase + q) * NC
        for j in range(NSEL // 16):
            idsg_v[pl.ds(j * 16, 16)] = idsl_v[pl.ds(j * 16, 16)] + gbase
        gc0 = pltpu.async_copy(d2t_hbm.at[idsg_v], gd2_v, semg)
        gc1 = pltpu.async_copy(itab_hbm.at[idsl_v.at[pl.ds(0, NSEL)]],
                               gidx_v, semg)
        gc0.wait()
        gc1.wait()
        for j in range(CAP // 16 + 1):
            cd_v[pl.ds(j * 16, 16)] = jnp.full((16,), INF, jnp.float32)
            ci_v[pl.ds(j * 16, 16)] = jnp.zeros((16,), jnp.int32)

        def stepB(j, off):
            row = gd2_v[j // 8, pl.ds((j % 8) * 16, 16)]
            msk = row <= tau16
            rank = _prefix16(msk, lane)
            n = rank[15]

            @pl.when(n > 0)
            def _():
                idxv = gidx_v[j // 8, pl.ds((j % 8) * 16, 16)]
                crow, cidx = _compact16([row, idxv], rank, lane, INF)
                offc = jnp.minimum(off, CAP)
                cd_v[pl.ds(offc, 16)] = crow
                ci_v[pl.ds(offc, 16)] = cidx

            return off + n

        lax.fori_loop(0, na * 8, stepB, jnp.int32(0))
        oc0 = pltpu.async_copy(cd_v.at[pl.ds(0, CAP)], cd_hbm.at[qbase + q],
                               semo)
        oc1 = pltpu.async_copy(ci_v.at[pl.ds(0, CAP)], ci_hbm.at[qbase + q],
                               semo)
        oc0.wait()
        oc1.wait()
        cp_cur = cp_next


def _run_k2(vmin_rows, d2t, itab):
    mesh = plsc.VectorSubcoreMesh(core_axis_name="c", subcore_axis_name="s")
    kern = functools.partial(
        pl.kernel,
        mesh=mesh,
        out_type=[
            jax.ShapeDtypeStruct((NQ, CAP), jnp.float32),
            jax.ShapeDtypeStruct((NQ, CAP), jnp.int32),
        ],
        scratch_types=[
            pltpu.VMEM((NVP,), jnp.float32),
            pltpu.VMEM((NVP,), jnp.float32),
            pltpu.VMEM((128,), jnp.int32),
            pltpu.VMEM((NSEL,), jnp.int32),
            pltpu.VMEM((NSEL, CL), jnp.float32),
            pltpu.VMEM((NSEL, CL), jnp.int32),
            pltpu.VMEM((CAP + 128,), jnp.float32),
            pltpu.VMEM((CAP + 128,), jnp.int32),
            pltpu.SemaphoreType.DMA,
            pltpu.SemaphoreType.DMA,
            pltpu.SemaphoreType.DMA,
            pltpu.SemaphoreType.DMA,
        ],
    )(_k2_body)
    return kern(vmin_rows, d2t, itab)


def _k3_body(cd_ref, ci_ref, gid_ref):
    b = pl.program_id(0)
    d2 = cd_ref[0]
    idx = ci_ref[0].astype(jnp.float32)
    valid = d2 < jnp.float32(1e38)
    dist = jnp.sqrt(d2)
    rank = jnp.zeros((AB, CAP), jnp.int32)
    for j in range(CAP):
        dj = dist[:, j:j + 1]
        ij = idx[:, j:j + 1]
        vj = valid[:, j:j + 1]
        less = jnp.logical_and(
            vj, jnp.logical_or(dj < dist,
                               jnp.logical_and(dj == dist, ij < idx)))
        rank = rank + jnp.where(less, 1, 0)
    rank = jnp.where(valid, rank, jnp.int32(10000))
    base = (b // (A // AB)) * M
    gid = idx.astype(jnp.int32) + base
    cols = []
    for k in range(K_LOCAL):
        sel = rank == k
        cols.append(jnp.sum(jnp.where(sel, gid, 0), axis=-1, keepdims=True))
    gid_ref[0] = jnp.concatenate(cols, axis=-1)


def _run_k3(cd, ci):
    return pl.pallas_call(
        _k3_body,
        grid=(NQ // AB,),
        in_specs=[
            pl.BlockSpec((1, AB, CAP), lambda i: (i, 0, 0)),
            pl.BlockSpec((1, AB, CAP), lambda i: (i, 0, 0)),
        ],
        out_specs=pl.BlockSpec((1, AB, K_LOCAL), lambda i: (i, 0, 0)),
        out_shape=jax.ShapeDtypeStruct((NQ // AB, AB, K_LOCAL), jnp.int32),
    )(cd, ci)


def _k4_body(tab_hbm, gid_hbm, out_hbm, idx_v, rows_v, sem, semo):
    info = plsc.get_sparse_core_info()
    nc_sc = info.num_cores
    wid = lax.axis_index("s") * nc_sc + lax.axis_index("c")
    rpw = (NQ * K_LOCAL) // (nc_sc * 16)
    base = wid * rpw
    pltpu.sync_copy(gid_hbm.at[pl.ds(base, rpw)], idx_v)
    copies = []
    nfull = rpw // 128
    for c in range(nfull):
        copies.append(pltpu.async_copy(
            tab_hbm.at[idx_v.at[pl.ds(c * 128, 128)]],
            rows_v.at[pl.ds(c * 128, 128), :], sem))
    rem = rpw - nfull * 128
    if rem:
        copies.append(pltpu.async_copy(
            tab_hbm.at[idx_v.at[pl.ds(nfull * 128, rem)]],
            rows_v.at[pl.ds(nfull * 128, rem), :], sem))
    for cp in copies:
        cp.wait()
    oc = pltpu.async_copy(rows_v, out_hbm.at[pl.ds(base, rpw), :], semo)
    oc.wait()


def _run_k4(tab, gid_flat):
    mesh = plsc.VectorSubcoreMesh(core_axis_name="c", subcore_axis_name="s")
    rpw = (NQ * K_LOCAL) // 32
    kern = functools.partial(
        pl.kernel,
        mesh=mesh,
        compiler_params=pltpu.CompilerParams(use_tc_tiling_on_sc=False),
        out_type=jax.ShapeDtypeStruct((NQ * K_LOCAL, 8), jnp.float32),
        scratch_types=[
            pltpu.VMEM((rpw,), jnp.int32),
            pltpu.VMEM((rpw, 8), jnp.float32),
            pltpu.SemaphoreType.DMA,
            pltpu.SemaphoreType.DMA,
        ],
    )(_k4_body)
    return kern(tab, gid_flat)


def _mm(x3, W_ref, b_ref, kdim):
    acc = jnp.broadcast_to(b_ref[0:1, :][None], x3.shape[:2] + (EMB,))
    for k in range(kdim):
        acc = acc + x3[:, :, k:k + 1] * W_ref[k:k + 1, :][None]
    return acc


def _rel_part(feat, nfeat, axs, ays, mref, Wp_ref, bp_ref, We_ref, be_ref,
              kemb):
    N = feat.shape[0]
    fx = feat[:, 0:1][None]
    fy = feat[:, 1:2][None]
    px = fx - axs
    py = fy - ays
    pos = jnp.concatenate(
        [px, py,
         jnp.broadcast_to(feat[:, 2:3][None], (AB, N, 1)),
         jnp.broadcast_to(feat[:, 3:4][None], (AB, N, 1))], axis=-1)
    emb = _mm(pos, Wp_ref, bp_ref, 4)
    base = _mm(feat[None, :, :kemb], We_ref, be_ref, kemb)
    emb = emb + base
    dist = jnp.sqrt(px * px + py * py)
    msk = jnp.logical_or(mref[0, 0:1, :][:, :, None] > 0.5,
                         dist > 1.0)[:, :, 0]
    return emb, msk.astype(jnp.float32)


def _k5_body(m2a_ref, agt_ref, agf_ref, li_ref, st_ref, wa_ref,
             lm_ref, sm_ref, wm_ref, am_ref,
             wmap_ref, bmap_ref, wm2a_ref, bm2a_ref,
             wlight_ref, blight_ref, wl2a_ref, bl2a_ref,
             wstop_ref, bstop_ref, ws2a_ref, bs2a_ref,
             wwalk_ref, bwalk_ref, ww2a_ref, bw2a_ref,
             wego_ref, bego_ref, wag_ref, bag_ref, wa2a_ref, ba2a_ref,
             m2a_o, l2a_o, s2a_o, w2a_o, a2a_o, aemb_o,
             mm_o, lm_o, sm_o, wm_o, am_o):
    j = pl.program_id(1)
    agt = agt_ref[0]
    agf = agf_ref[0]
    axs = agt[:, 0:1][:, None, :]
    ays = agt[:, 1:2][:, None, :]

    f3 = m2a_ref[0].reshape(AB, K_LOCAL, 8)
    px = f3[:, :, 0:1] - axs
    py = f3[:, :, 1:2] - ays
    e0x = f3[:, :, 0:1] - agf[0, 0]
    e0y = f3[:, :, 1:2] - agf[0, 1]
    ego_d = jnp.sqrt(e0x * e0x + e0y * e0y)
    gmask = f3[:, :, 7:8]
    unobs = jnp.logical_or(ego_d > 1.0, gmask > 0.5)
    obs = jnp.where(unobs, 0.0, 1.0)
    pos4 = jnp.concatenate([px, py, f3[:, :, 2:3], f3[:, :, 3:4]], axis=-1)
    m2a = _mm(pos4 * obs, wm2a_ref, bm2a_ref, 4) + _mm(
        f3[:, :, :7], wmap_ref, bmap_ref, 7)
    mdist = jnp.sqrt(px * px + py * py)
    m2a_o[0] = m2a
    mm_o[0] = jnp.logical_or(gmask > 0.5,
                             mdist > 1.0)[:, :, 0].astype(jnp.float32)

    l2a, lmsk = _rel_part(li_ref[0], 8, axs, ays, lm_ref, wl2a_ref, bl2a_ref,
                          wlight_ref, blight_ref, 8)
    l2a_o[0] = l2a
    lm_o[0] = lmsk
    s2a, smsk = _rel_part(st_ref[0], 5, axs, ays, sm_ref, ws2a_ref, bs2a_ref,
                          wstop_ref, bstop_ref, 5)
    s2a_o[0] = s2a
    sm_o[0] = smsk
    w2a, wmsk = _rel_part(wa_ref[0], 6, axs, ays, wm_ref, ww2a_ref, bw2a_ref,
                          wwalk_ref, bwalk_ref, 6)
    w2a_o[0] = w2a
    wm_o[0] = wmsk

    gx = agf[:, 0:1][None]
    gy = agf[:, 1:2][None]
    apx = gx - axs
    apy = gy - ays
    apos = jnp.concatenate(
        [apx, apy,
         jnp.broadcast_to(agf[:, 2:3][None], (AB, A, 1)),
         jnp.broadcast_to(agf[:, 3:4][None], (AB, A, 1))], axis=-1)
    a2a_o[0] = _mm(apos, wa2a_ref, ba2a_ref, 4)
    adist = jnp.sqrt(apx * apx + apy * apy)
    am_o[0] = jnp.logical_or(am_ref[0, 0:1, :][:, :, None] > 0.5,
                             adist > 1.0)[:, :, 0].astype(jnp.float32)

    ego = _mm(agt[None], wego_ref, bego_ref, 7)[0]
    oth = _mm(agt[None], wag_ref, bag_ref, 7)[0]
    rows = lax.broadcasted_iota(jnp.int32, (AB, 1), 0) + j * AB
    aemb_o[0] = jnp.where(rows == 0, ego, oth)


def _run_k5(m2a_rows, agents, lights, stops, walkers,
            lmask_f, smask_f, wmask_f, amask_f, weights):
    L, S, Wn = 16, 32, 64

    def wspec(w):
        return pl.BlockSpec(w.shape, lambda b, j: (0,) * w.ndim)

    win = []
    wargs = []
    for w in weights:
        w2 = w.reshape(1, EMB) if w.ndim == 1 else w
        wargs.append(w2)
        win.append(wspec(w2))

    def bo(shape):
        nd = len(shape)
        return pl.BlockSpec((1, 1) + shape,
                            lambda b, j: (b, j) + (0,) * nd)

    outs = [
        (K_LOCAL, EMB), (L, EMB), (S, EMB), (Wn, EMB), (A, EMB), (EMB,),
        (K_LOCAL,), (L,), (S,), (Wn,), (A,),
    ]
    return pl.pallas_call(
        _k5_body,
        grid=(B, A // AB),
        in_specs=[
            pl.BlockSpec((1, AB * K_LOCAL, 8),
                         lambda b, j: (b * (A // AB) + j, 0, 0)),
            pl.BlockSpec((1, AB, 7), lambda b, j: (b, j, 0)),
            pl.BlockSpec((1, A, 7), lambda b, j: (b, 0, 0)),
            pl.BlockSpec((1, L, 8), lambda b, j: (b, 0, 0)),
            pl.BlockSpec((1, S, 5), lambda b, j: (b, 0, 0)),
            pl.BlockSpec((1, Wn, 6), lambda b, j: (b, 0, 0)),
            pl.BlockSpec((1, 1, L), lambda b, j: (b, 0, 0)),
            pl.BlockSpec((1, 1, S), lambda b, j: (b, 0, 0)),
            pl.BlockSpec((1, 1, Wn), lambda b, j: (b, 0, 0)),
            pl.BlockSpec((1, 1, A), lambda b, j: (b, 0, 0)),
        ] + win,
        out_specs=[
            pl.BlockSpec((1, AB) + sh, (lambda b, j, n=len(sh): (b, j) + (0,) * n))
            for sh in outs
        ],
        out_shape=[
            jax.ShapeDtypeStruct((B, A) + sh, jnp.float32) for sh in outs
        ],
    )(m2a_rows.reshape(NQ // AB, AB * K_LOCAL, 8), agents, agents,
      lights, stops, walkers, lmask_f, smask_f, wmask_f, amask_f, *wargs)


def kernel(map_features, map_masks, light_features, light_masks, stop_features,
           stop_masks, walker_features, walker_masks, agents_features,
           agents_masks, W_map, b_map, W_m2a, b_m2a, W_light, b_light, W_l2a,
           b_l2a, W_stop, b_stop, W_s2a, b_s2a, W_walk, b_walk, W_w2a, b_w2a,
           W_ego, b_ego, W_ag, b_ag, W_a2a, b_a2a):
    mapt = jnp.transpose(map_features[:, :, :2], (0, 2, 1))
    mapt = jnp.pad(mapt, ((0, 0), (0, 0), (0, MP - M)))
    maskf = map_masks.astype(jnp.float32)[:, None, :]
    maskt = jnp.pad(maskf, ((0, 0), (0, 0), (0, MP - M)),
                    constant_values=1.0)
    tab = jnp.concatenate(
        [map_features, map_masks.astype(jnp.float32)[:, :, None]],
        axis=-1).reshape(B * M, 8)

    d2, vmin = _run_k1(mapt, maskt, agents_features)
    d2t = d2.reshape(NQ * NC, CL)
    vmin_rows = vmin.reshape(NQ, NVP)

    itab = jnp.arange(NC * CL, dtype=jnp.int32).reshape(NC, CL)
    cd, ci = _run_k2(vmin_rows, d2t, itab)
    cd3 = cd.reshape(NQ // AB, AB, CAP)
    ci3 = ci.reshape(NQ // AB, AB, CAP)

    gid = _run_k3(cd3, ci3)
    gid_flat = gid.reshape(NQ * K_LOCAL)

    rows = _run_k4(tab, gid_flat)
    m2a_rows = rows

    weights = (W_map, b_map, W_m2a, b_m2a, W_light, b_light, W_l2a, b_l2a,
               W_stop, b_stop, W_s2a, b_s2a, W_walk, b_walk, W_w2a, b_w2a,
               W_ego, b_ego, W_ag, b_ag, W_a2a, b_a2a)
    (m2a_e, l2a_e, s2a_e, w2a_e, a2a_emb, agents_emb,
     mm_f, lm_f, sm_f, wm_f, am_f) = _run_k5(
        m2a_rows, agents_features, light_features, stop_features,
        walker_features,
        light_masks.astype(jnp.float32)[:, None, :],
        stop_masks.astype(jnp.float32)[:, None, :],
        walker_masks.astype(jnp.float32)[:, None, :],
        agents_masks.astype(jnp.float32)[:, None, :],
        weights)
    scene_emb = jnp.concatenate([m2a_e, l2a_e, s2a_e, w2a_e], axis=2)
    scene_masks = jnp.concatenate([mm_f, lm_f, sm_f, wm_f], axis=2) > 0.5
    return (scene_emb, scene_masks, agents_emb, a2a_emb, am_f > 0.5)


# speedup vs baseline: 4.5188x; 1.0349x over previous
import functools

import jax
import jax.numpy as jnp
from jax import lax
from jax.experimental import pallas as pl
from jax.experimental.pallas import tpu as pltpu
from jax.experimental.pallas import tpu_sc as plsc

EMB = 128
K_LOCAL = 50
B, M, A = 16, 20000, 64
CL = 128
NC = 160
MP = NC * CL
CAP = 128
AB = 8
NQ = B * A
INF = float("inf")


NVP = NC + 16
NSEL = 64


def _k1_body(mapt_ref, maskt_ref, ag_ref, d2_ref, vmin_ref):
    qx = ag_ref[0, :, 0:1]
    qy = ag_ref[0, :, 1:2]
    mx = mapt_ref[0, 0:1, :]
    my = mapt_ref[0, 1:2, :]
    mk = maskt_ref[0, 0:1, :]
    dx = mx - qx
    dy = my - qy
    d2 = dx * dx + dy * dy
    d2 = jnp.where(mk > 0.5, INF, d2)
    d2_ref[0] = d2
    vm = jnp.min(d2.reshape(AB, NC, CL), axis=-1)

    def step(_, t):
        masked = jnp.where(vm > t, vm, INF)
        return jnp.min(masked, axis=-1, keepdims=True)

    t = lax.fori_loop(0, K_LOCAL, step, jnp.full((AB, 1), -INF, jnp.float32))
    vmin_ref[0, :, 0:NC] = vm
    vmin_ref[0, :, NC:NVP] = jnp.broadcast_to(t, (AB, 16))


def _run_k1(mapt, maskt, agents):
    return pl.pallas_call(
        _k1_body,
        grid=(B, A // AB),
        in_specs=[
            pl.BlockSpec((1, 2, MP), lambda b, j: (b, 0, 0)),
            pl.BlockSpec((1, 1, MP), lambda b, j: (b, 0, 0)),
            pl.BlockSpec((1, AB, 7), lambda b, j: (b, j, 0)),
        ],
        out_specs=[
            pl.BlockSpec((1, AB, MP), lambda b, j: (b, j, 0)),
            pl.BlockSpec((1, AB, NVP), lambda b, j: (b, j, 0)),
        ],
        out_shape=[
            jax.ShapeDtypeStruct((B, A, MP), jnp.float32),
            jax.ShapeDtypeStruct((B, A, NVP), jnp.float32),
        ],
    )(mapt, maskt, agents)


def _gather16(x, idx):
    return x.at[idx].get(mode="promise_in_bounds")


def _prefix16(msk, lane):
    rank = jnp.where(msk, 1, 0).astype(jnp.int32)
    for k in (1, 2, 4, 8):
        sh = _gather16(rank, jnp.maximum(lane - k, 0))
        rank = rank + jnp.where(lane >= k, sh, 0)
    return rank


def _compact16(vals, rank, lane, pad):
    tgt = lane + 1
    p = jnp.where(rank[7] < tgt, 8, 0).astype(jnp.int32)
    for step in (4, 2, 1):
        r = _gather16(rank, p + (step - 1))
        p = p + jnp.where(r < tgt, step, 0)
    okm = tgt <= rank[15]
    out0 = jnp.where(okm, _gather16(vals[0], p), pad)
    return [out0] + [_gather16(v, p) for v in vals[1:]]


def _k2_body(vmin_hbm, d2t_hbm, itab_hbm, cd_hbm, ci_hbm,
             vrow0, vrow1, idsl_v, idsg_v, gd2_v, gidx_v, cd_v, ci_v,
             sem0, sem1, semg, semo):
    info = plsc.get_sparse_core_info()
    nc_sc = info.num_cores
    wid = lax.axis_index("s") * nc_sc + lax.axis_index("c")
    qpw = NQ // (nc_sc * 16)
    qbase = wid * qpw
    lane = lax.iota(jnp.int32, 16)
    l15 = jnp.full((16,), 15, jnp.int32)
    vrows = (vrow0, vrow1)
    sems = (sem0, sem1)
    cp_cur = pltpu.async_copy(vmin_hbm.at[qbase], vrows[0], sems[0])

    for q in range(qpw):
        cp_next = None
        if q + 1 < qpw:
            cp_next = pltpu.async_copy(vmin_hbm.at[qbase + q + 1],
                                       vrows[(q + 1) % 2], sems[(q + 1) % 2])
        cp_cur.wait()
        vrow = vrows[q % 2]
        tau16 = vrow[pl.ds(NC, 16)]

        for j in range(NSEL // 16):
            idsl_v[pl.ds(j * 16, 16)] = jnp.full((16,), NC - 1, jnp.int32)

        def stepA(j, off):
            vm = vrow[pl.ds(j * 16, 16)]
            msk = vm <= tau16
            rank = _prefix16(msk, lane)
            n = rank[15]

            @pl.when(n > 0)
            def _():
                (ids,) = _compact16([lane + j * 16], rank, lane, NC - 1)
                offc = jnp.minimum(off, NSEL)
                idsl_v[pl.ds(offc, 16)] = ids

            return off + n

        na = lax.fori_loop(0, NC // 16, stepA, jnp.int32(0))
        na = jnp.minimum(na, NSEL)

        gbase = (qbase + q) * NC
        for j in range(NSEL // 16):
            idsg_v[pl.ds(j * 16, 16)] = idsl_v[pl.ds(j * 16, 16)] + gbase
        gc0 = pltpu.async_copy(d2t_hbm.at[idsg_v], gd2_v, semg)
        gc1 = pltpu.async_copy(itab_hbm.at[idsl_v.at[pl.ds(0, NSEL)]],
                               gidx_v, semg)
        gc0.wait()
        gc1.wait()
        for j in range(CAP // 16 + 1):
            cd_v[pl.ds(j * 16, 16)] = jnp.full((16,), INF, jnp.float32)
            ci_v[pl.ds(j * 16, 16)] = jnp.zeros((16,), jnp.int32)

        def stepB(j, off):
            row = gd2_v[j // 8, pl.ds((j % 8) * 16, 16)]
            msk = row <= tau16
            rank = _prefix16(msk, lane)
            n = rank[15]

            @pl.when(n > 0)
            def _():
                idxv = gidx_v[j // 8, pl.ds((j % 8) * 16, 16)]
                crow, cidx = _compact16([row, idxv], rank, lane, INF)
                offc = jnp.minimum(off, CAP)
                cd_v[pl.ds(offc, 16)] = crow
                ci_v[pl.ds(offc, 16)] = cidx

            return off + n

        lax.fori_loop(0, na * 8, stepB, jnp.int32(0))
        oc0 = pltpu.async_copy(cd_v.at[pl.ds(0, CAP)], cd_hbm.at[qbase + q],
                               semo)
        oc1 = pltpu.async_copy(ci_v.at[pl.ds(0, CAP)], ci_hbm.at[qbase + q],
                               semo)
        oc0.wait()
        oc1.wait()
        cp_cur = cp_next


def _run_k2(vmin_rows, d2t, itab):
    mesh = plsc.VectorSubcoreMesh(core_axis_name="c", subcore_axis_name="s")
    kern = functools.partial(
        pl.kernel,
        mesh=mesh,
        out_type=[
            jax.ShapeDtypeStruct((NQ, CAP), jnp.float32),
            jax.ShapeDtypeStruct((NQ, CAP), jnp.int32),
        ],
        scratch_types=[
            pltpu.VMEM((NVP,), jnp.float32),
            pltpu.VMEM((NVP,), jnp.float32),
            pltpu.VMEM((128,), jnp.int32),
            pltpu.VMEM((NSEL,), jnp.int32),
            pltpu.VMEM((NSEL, CL), jnp.float32),
            pltpu.VMEM((NSEL, CL), jnp.int32),
            pltpu.VMEM((CAP + 128,), jnp.float32),
            pltpu.VMEM((CAP + 128,), jnp.int32),
            pltpu.SemaphoreType.DMA,
            pltpu.SemaphoreType.DMA,
            pltpu.SemaphoreType.DMA,
            pltpu.SemaphoreType.DMA,
        ],
    )(_k2_body)
    return kern(vmin_rows, d2t, itab)


def _k3_body(cd_ref, ci_ref, gid_ref):
    b = pl.program_id(0)
    d2 = cd_ref[0]
    idx = ci_ref[0].astype(jnp.float32)
    valid = d2 < jnp.float32(1e38)
    dist = jnp.sqrt(d2)
    rank = jnp.zeros((AB, CAP), jnp.int32)
    for j in range(CAP):
        dj = dist[:, j:j + 1]
        ij = idx[:, j:j + 1]
        vj = valid[:, j:j + 1]
        less = jnp.logical_and(
            vj, jnp.logical_or(dj < dist,
                               jnp.logical_and(dj == dist, ij < idx)))
        rank = rank + jnp.where(less, 1, 0)
    rank = jnp.where(valid, rank, jnp.int32(10000))
    base = (b // (A // AB)) * M
    gid = idx.astype(jnp.int32) + base
    cols = []
    for k in range(K_LOCAL):
        sel = rank == k
        cols.append(jnp.sum(jnp.where(sel, gid, 0), axis=-1, keepdims=True))
    gid_ref[0] = jnp.concatenate(cols, axis=-1)


def _run_k3(cd, ci):
    return pl.pallas_call(
        _k3_body,
        grid=(NQ // AB,),
        in_specs=[
            pl.BlockSpec((1, AB, CAP), lambda i: (i, 0, 0)),
            pl.BlockSpec((1, AB, CAP), lambda i: (i, 0, 0)),
        ],
        out_specs=pl.BlockSpec((1, AB, K_LOCAL), lambda i: (i, 0, 0)),
        out_shape=jax.ShapeDtypeStruct((NQ // AB, AB, K_LOCAL), jnp.int32),
    )(cd, ci)


def _k4_body(tab_hbm, gid_hbm, out_hbm, idx_v, rows_v, sem, semo):
    info = plsc.get_sparse_core_info()
    nc_sc = info.num_cores
    wid = lax.axis_index("s") * nc_sc + lax.axis_index("c")
    rpw = (NQ * K_LOCAL) // (nc_sc * 16)
    base = wid * rpw
    pltpu.sync_copy(gid_hbm.at[pl.ds(base, rpw)], idx_v)
    copies = []
    nfull = rpw // 128
    for c in range(nfull):
        copies.append(pltpu.async_copy(
            tab_hbm.at[idx_v.at[pl.ds(c * 128, 128)]],
            rows_v.at[pl.ds(c * 128, 128), :], sem))
    rem = rpw - nfull * 128
    if rem:
        copies.append(pltpu.async_copy(
            tab_hbm.at[idx_v.at[pl.ds(nfull * 128, rem)]],
            rows_v.at[pl.ds(nfull * 128, rem), :], sem))
    for cp in copies:
        cp.wait()
    oc = pltpu.async_copy(rows_v, out_hbm.at[pl.ds(base, rpw), :], semo)
    oc.wait()


def _run_k4(tab, gid_flat):
    mesh = plsc.VectorSubcoreMesh(core_axis_name="c", subcore_axis_name="s")
    rpw = (NQ * K_LOCAL) // 32
    kern = functools.partial(
        pl.kernel,
        mesh=mesh,
        compiler_params=pltpu.CompilerParams(use_tc_tiling_on_sc=False),
        out_type=jax.ShapeDtypeStruct((NQ * K_LOCAL, 8), jnp.float32),
        scratch_types=[
            pltpu.VMEM((rpw,), jnp.int32),
            pltpu.VMEM((rpw, 8), jnp.float32),
            pltpu.SemaphoreType.DMA,
            pltpu.SemaphoreType.DMA,
        ],
    )(_k4_body)
    return kern(tab, gid_flat)


def _mm(x3, W_ref, b_ref, kdim):
    acc = jnp.broadcast_to(b_ref[0:1, :][None], x3.shape[:2] + (EMB,))
    for k in range(kdim):
        acc = acc + x3[:, :, k:k + 1] * W_ref[k:k + 1, :][None]
    return acc


def _rel_part(feat, nfeat, axs, ays, mref, Wp_ref, bp_ref, We_ref, be_ref,
              kemb):
    N = feat.shape[0]
    fx = feat[:, 0:1][None]
    fy = feat[:, 1:2][None]
    px = fx - axs
    py = fy - ays
    pos = jnp.concatenate(
        [px, py,
         jnp.broadcast_to(feat[:, 2:3][None], (AB, N, 1)),
         jnp.broadcast_to(feat[:, 3:4][None], (AB, N, 1))], axis=-1)
    emb = _mm(pos, Wp_ref, bp_ref, 4)
    base = _mm(feat[None, :, :kemb], We_ref, be_ref, kemb)
    emb = emb + base
    dist = jnp.sqrt(px * px + py * py)
    msk = jnp.logical_or(mref[0, 0:1, :][:, :, None] > 0.5,
                         dist > 1.0)[:, :, 0]
    return emb, msk.astype(jnp.float32)


def _k5_body(m2a_ref, agt_ref, agf_ref, li_ref, st_ref, wa_ref,
             lm_ref, sm_ref, wm_ref, am_ref,
             wmap_ref, bmap_ref, wm2a_ref, bm2a_ref,
             wlight_ref, blight_ref, wl2a_ref, bl2a_ref,
             wstop_ref, bstop_ref, ws2a_ref, bs2a_ref,
             wwalk_ref, bwalk_ref, ww2a_ref, bw2a_ref,
             wego_ref, bego_ref, wag_ref, bag_ref, wa2a_ref, ba2a_ref,
             scene_o, a2a_o, aemb_o, smask_o, am_o):
    j = pl.program_id(1)
    agt = agt_ref[0]
    agf = agf_ref[0]
    axs = agt[:, 0:1][:, None, :]
    ays = agt[:, 1:2][:, None, :]

    f3 = m2a_ref[0].reshape(AB, K_LOCAL, 8)
    px = f3[:, :, 0:1] - axs
    py = f3[:, :, 1:2] - ays
    e0x = f3[:, :, 0:1] - agf[0, 0]
    e0y = f3[:, :, 1:2] - agf[0, 1]
    ego_d = jnp.sqrt(e0x * e0x + e0y * e0y)
    gmask = f3[:, :, 7:8]
    unobs = jnp.logical_or(ego_d > 1.0, gmask > 0.5)
    obs = jnp.where(unobs, 0.0, 1.0)
    pos4 = jnp.concatenate([px, py, f3[:, :, 2:3], f3[:, :, 3:4]], axis=-1)
    m2a = _mm(pos4 * obs, wm2a_ref, bm2a_ref, 4) + _mm(
        f3[:, :, :7], wmap_ref, bmap_ref, 7)
    mdist = jnp.sqrt(px * px + py * py)
    scene_o[0, :, 0:K_LOCAL, :] = m2a
    mmsk = jnp.logical_or(gmask > 0.5,
                          mdist > 1.0)[:, :, 0].astype(jnp.float32)

    l2a, lmsk = _rel_part(li_ref[0], 8, axs, ays, lm_ref, wl2a_ref, bl2a_ref,
                          wlight_ref, blight_ref, 8)
    scene_o[0, :, K_LOCAL:K_LOCAL + 16, :] = l2a
    s2a, smsk = _rel_part(st_ref[0], 5, axs, ays, sm_ref, ws2a_ref, bs2a_ref,
                          wstop_ref, bstop_ref, 5)
    scene_o[0, :, K_LOCAL + 16:K_LOCAL + 48, :] = s2a
    w2a, wmsk = _rel_part(wa_ref[0], 6, axs, ays, wm_ref, ww2a_ref, bw2a_ref,
                          wwalk_ref, bwalk_ref, 6)
    scene_o[0, :, K_LOCAL + 48:K_LOCAL + 112, :] = w2a
    smask_o[0] = jnp.concatenate([mmsk, lmsk, smsk, wmsk], axis=1)

    gx = agf[:, 0:1][None]
    gy = agf[:, 1:2][None]
    apx = gx - axs
    apy = gy - ays
    apos = jnp.concatenate(
        [apx, apy,
         jnp.broadcast_to(agf[:, 2:3][None], (AB, A, 1)),
         jnp.broadcast_to(agf[:, 3:4][None], (AB, A, 1))], axis=-1)
    a2a_o[0] = _mm(apos, wa2a_ref, ba2a_ref, 4)
    adist = jnp.sqrt(apx * apx + apy * apy)
    am_o[0] = jnp.logical_or(am_ref[0, 0:1, :][:, :, None] > 0.5,
                             adist > 1.0)[:, :, 0].astype(jnp.float32)

    ego = _mm(agt[None], wego_ref, bego_ref, 7)[0]
    oth = _mm(agt[None], wag_ref, bag_ref, 7)[0]
    rows = lax.broadcasted_iota(jnp.int32, (AB, 1), 0) + j * AB
    aemb_o[0] = jnp.where(rows == 0, ego, oth)


def _run_k5(m2a_rows, agents, lights, stops, walkers,
            lmask_f, smask_f, wmask_f, amask_f, weights):
    L, S, Wn = 16, 32, 64

    def wspec(w):
        return pl.BlockSpec(w.shape, lambda b, j: (0,) * w.ndim)

    win = []
    wargs = []
    for w in weights:
        w2 = w.reshape(1, EMB) if w.ndim == 1 else w
        wargs.append(w2)
        win.append(wspec(w2))

    def bo(shape):
        nd = len(shape)
        return pl.BlockSpec((1, 1) + shape,
                            lambda b, j: (b, j) + (0,) * nd)

    NS = K_LOCAL + L + S + Wn
    outs = [(NS, EMB), (A, EMB), (EMB,), (NS,), (A,)]
    return pl.pallas_call(
        _k5_body,
        grid=(B, A // AB),
        in_specs=[
            pl.BlockSpec((1, AB * K_LOCAL, 8),
                         lambda b, j: (b * (A // AB) + j, 0, 0)),
            pl.BlockSpec((1, AB, 7), lambda b, j: (b, j, 0)),
            pl.BlockSpec((1, A, 7), lambda b, j: (b, 0, 0)),
            pl.BlockSpec((1, L, 8), lambda b, j: (b, 0, 0)),
            pl.BlockSpec((1, S, 5), lambda b, j: (b, 0, 0)),
            pl.BlockSpec((1, Wn, 6), lambda b, j: (b, 0, 0)),
            pl.BlockSpec((1, 1, L), lambda b, j: (b, 0, 0)),
            pl.BlockSpec((1, 1, S), lambda b, j: (b, 0, 0)),
            pl.BlockSpec((1, 1, Wn), lambda b, j: (b, 0, 0)),
            pl.BlockSpec((1, 1, A), lambda b, j: (b, 0, 0)),
        ] + win,
        out_specs=[
            pl.BlockSpec((1, AB) + sh, (lambda b, j, n=len(sh): (b, j) + (0,) * n))
            for sh in outs
        ],
        out_shape=[
            jax.ShapeDtypeStruct((B, A) + sh, jnp.float32) for sh in outs
        ],
    )(m2a_rows.reshape(NQ // AB, AB * K_LOCAL, 8), agents, agents,
      lights, stops, walkers, lmask_f, smask_f, wmask_f, amask_f, *wargs)


def kernel(map_features, map_masks, light_features, light_masks, stop_features,
           stop_masks, walker_features, walker_masks, agents_features,
           agents_masks, W_map, b_map, W_m2a, b_m2a, W_light, b_light, W_l2a,
           b_l2a, W_stop, b_stop, W_s2a, b_s2a, W_walk, b_walk, W_w2a, b_w2a,
           W_ego, b_ego, W_ag, b_ag, W_a2a, b_a2a):
    mapt = jnp.transpose(map_features[:, :, :2], (0, 2, 1))
    mapt = jnp.pad(mapt, ((0, 0), (0, 0), (0, MP - M)))
    maskf = map_masks.astype(jnp.float32)[:, None, :]
    maskt = jnp.pad(maskf, ((0, 0), (0, 0), (0, MP - M)),
                    constant_values=1.0)
    tab = jnp.concatenate(
        [map_features, map_masks.astype(jnp.float32)[:, :, None]],
        axis=-1).reshape(B * M, 8)

    d2, vmin = _run_k1(mapt, maskt, agents_features)
    d2t = d2.reshape(NQ * NC, CL)
    vmin_rows = vmin.reshape(NQ, NVP)

    itab = jnp.arange(NC * CL, dtype=jnp.int32).reshape(NC, CL)
    cd, ci = _run_k2(vmin_rows, d2t, itab)
    cd3 = cd.reshape(NQ // AB, AB, CAP)
    ci3 = ci.reshape(NQ // AB, AB, CAP)

    gid = _run_k3(cd3, ci3)
    gid_flat = gid.reshape(NQ * K_LOCAL)

    rows = _run_k4(tab, gid_flat)
    m2a_rows = rows

    weights = (W_map, b_map, W_m2a, b_m2a, W_light, b_light, W_l2a, b_l2a,
               W_stop, b_stop, W_s2a, b_s2a, W_walk, b_walk, W_w2a, b_w2a,
               W_ego, b_ego, W_ag, b_ag, W_a2a, b_a2a)
    (scene_emb, a2a_emb, agents_emb, smask_f, am_f) = _run_k5(
        m2a_rows, agents_features, light_features, stop_features,
        walker_features,
        light_masks.astype(jnp.float32)[:, None, :],
        stop_masks.astype(jnp.float32)[:, None, :],
        walker_masks.astype(jnp.float32)[:, None, :],
        agents_masks.astype(jnp.float32)[:, None, :],
        weights)
    return (scene_emb, smask_f > 0.5, agents_emb, a2a_emb, am_f > 0.5)


# speedup vs baseline: 4.6022x; 1.0185x over previous
import functools

import jax
import jax.numpy as jnp
from jax import lax
from jax.experimental import pallas as pl
from jax.experimental.pallas import tpu as pltpu
from jax.experimental.pallas import tpu_sc as plsc

EMB = 128
K_LOCAL = 50
B, M, A = 16, 20000, 64
CL = 128
NC = 160
MP = NC * CL
CAP = 128
AB = 8
NQ = B * A
INF = float("inf")


NVP = NC + 16
NSEL = 64


def _k1_body(mapt_ref, maskt_ref, ag_ref, d2_ref, vmin_ref):
    qx = ag_ref[0, :, 0:1]
    qy = ag_ref[0, :, 1:2]
    mx = mapt_ref[0, 0:1, :]
    my = mapt_ref[0, 1:2, :]
    mk = maskt_ref[0, 0:1, :]
    dx = mx - qx
    dy = my - qy
    d2 = dx * dx + dy * dy
    d2 = jnp.where(mk > 0.5, INF, d2)
    d2_ref[...] = d2.reshape(AB * NC, CL)
    vm = jnp.min(d2.reshape(AB, NC, CL), axis=-1)

    def step(_, t):
        masked = jnp.where(vm > t, vm, INF)
        return jnp.min(masked, axis=-1, keepdims=True)

    t = lax.fori_loop(0, K_LOCAL, step, jnp.full((AB, 1), -INF, jnp.float32))
    vmin_ref[0, :, 0:NC] = vm
    vmin_ref[0, :, NC:NVP] = jnp.broadcast_to(t, (AB, 16))


def _run_k1(mapt, maskt, agents):
    return pl.pallas_call(
        _k1_body,
        grid=(B, A // AB),
        in_specs=[
            pl.BlockSpec((1, 2, MP), lambda b, j: (b, 0, 0)),
            pl.BlockSpec((1, 1, MP), lambda b, j: (b, 0, 0)),
            pl.BlockSpec((1, AB, 7), lambda b, j: (b, j, 0)),
        ],
        out_specs=[
            pl.BlockSpec((AB * NC, CL),
                         lambda b, j: (b * (A // AB) + j, 0)),
            pl.BlockSpec((1, AB, NVP), lambda b, j: (b, j, 0)),
        ],
        out_shape=[
            jax.ShapeDtypeStruct((NQ * NC, CL), jnp.float32),
            jax.ShapeDtypeStruct((B, A, NVP), jnp.float32),
        ],
    )(mapt, maskt, agents)


def _gather16(x, idx):
    return x.at[idx].get(mode="promise_in_bounds")


def _prefix16(msk, lane):
    rank = jnp.where(msk, 1, 0).astype(jnp.int32)
    for k in (1, 2, 4, 8):
        sh = _gather16(rank, jnp.maximum(lane - k, 0))
        rank = rank + jnp.where(lane >= k, sh, 0)
    return rank


def _compact16(vals, rank, lane, pad):
    tgt = lane + 1
    p = jnp.where(rank[7] < tgt, 8, 0).astype(jnp.int32)
    for step in (4, 2, 1):
        r = _gather16(rank, p + (step - 1))
        p = p + jnp.where(r < tgt, step, 0)
    okm = tgt <= rank[15]
    out0 = jnp.where(okm, _gather16(vals[0], p), pad)
    return [out0] + [_gather16(v, p) for v in vals[1:]]


def _k2_body(vmin_hbm, d2t_hbm, itab_hbm, cd_hbm, ci_hbm,
             vrow0, vrow1, idsl_v, idsg_v, gd2_v, gidx_v, cd_v, ci_v,
             sem0, sem1, semg, semo):
    info = plsc.get_sparse_core_info()
    nc_sc = info.num_cores
    wid = lax.axis_index("s") * nc_sc + lax.axis_index("c")
    qpw = NQ // (nc_sc * 16)
    qbase = wid * qpw
    lane = lax.iota(jnp.int32, 16)
    l15 = jnp.full((16,), 15, jnp.int32)
    vrows = (vrow0, vrow1)
    sems = (sem0, sem1)
    cp_cur = pltpu.async_copy(vmin_hbm.at[qbase], vrows[0], sems[0])

    for q in range(qpw):
        cp_next = None
        if q + 1 < qpw:
            cp_next = pltpu.async_copy(vmin_hbm.at[qbase + q + 1],
                                       vrows[(q + 1) % 2], sems[(q + 1) % 2])
        cp_cur.wait()
        vrow = vrows[q % 2]
        tau16 = vrow[pl.ds(NC, 16)]

        for j in range(NSEL // 16):
            idsl_v[pl.ds(j * 16, 16)] = jnp.full((16,), NC - 1, jnp.int32)

        def stepA(j, off):
            vm = vrow[pl.ds(j * 16, 16)]
            msk = vm <= tau16
            rank = _prefix16(msk, lane)
            n = rank[15]

            @pl.when(n > 0)
            def _():
                (ids,) = _compact16([lane + j * 16], rank, lane, NC - 1)
                offc = jnp.minimum(off, NSEL)
                idsl_v[pl.ds(offc, 16)] = ids

            return off + n

        na = lax.fori_loop(0, NC // 16, stepA, jnp.int32(0))
        na = jnp.minimum(na, NSEL)

        gbase = (qbase + q) * NC
        for j in range(NSEL // 16):
            idsg_v[pl.ds(j * 16, 16)] = idsl_v[pl.ds(j * 16, 16)] + gbase
        gc0 = pltpu.async_copy(d2t_hbm.at[idsg_v], gd2_v, semg)
        gc1 = pltpu.async_copy(itab_hbm.at[idsl_v.at[pl.ds(0, NSEL)]],
                               gidx_v, semg)
        gc0.wait()
        gc1.wait()
        for j in range(CAP // 16 + 1):
            cd_v[pl.ds(j * 16, 16)] = jnp.full((16,), INF, jnp.float32)
            ci_v[pl.ds(j * 16, 16)] = jnp.zeros((16,), jnp.int32)

        def stepB(j, off):
            row = gd2_v[j // 8, pl.ds((j % 8) * 16, 16)]
            msk = row <= tau16
            rank = _prefix16(msk, lane)
            n = rank[15]

            @pl.when(n > 0)
            def _():
                idxv = gidx_v[j // 8, pl.ds((j % 8) * 16, 16)]
                crow, cidx = _compact16([row, idxv], rank, lane, INF)
                offc = jnp.minimum(off, CAP)
                cd_v[pl.ds(offc, 16)] = crow
                ci_v[pl.ds(offc, 16)] = cidx

            return off + n

        lax.fori_loop(0, na * 8, stepB, jnp.int32(0))
        oc0 = pltpu.async_copy(cd_v.at[pl.ds(0, CAP)], cd_hbm.at[qbase + q],
                               semo)
        oc1 = pltpu.async_copy(ci_v.at[pl.ds(0, CAP)], ci_hbm.at[qbase + q],
                               semo)
        oc0.wait()
        oc1.wait()
        cp_cur = cp_next


def _run_k2(vmin_rows, d2t, itab):
    mesh = plsc.VectorSubcoreMesh(core_axis_name="c", subcore_axis_name="s")
    kern = functools.partial(
        pl.kernel,
        mesh=mesh,
        out_type=[
            jax.ShapeDtypeStruct((NQ, CAP), jnp.float32),
            jax.ShapeDtypeStruct((NQ, CAP), jnp.int32),
        ],
        scratch_types=[
            pltpu.VMEM((NVP,), jnp.float32),
            pltpu.VMEM((NVP,), jnp.float32),
            pltpu.VMEM((128,), jnp.int32),
            pltpu.VMEM((NSEL,), jnp.int32),
            pltpu.VMEM((NSEL, CL), jnp.float32),
            pltpu.VMEM((NSEL, CL), jnp.int32),
            pltpu.VMEM((CAP + 128,), jnp.float32),
            pltpu.VMEM((CAP + 128,), jnp.int32),
            pltpu.SemaphoreType.DMA,
            pltpu.SemaphoreType.DMA,
            pltpu.SemaphoreType.DMA,
            pltpu.SemaphoreType.DMA,
        ],
    )(_k2_body)
    return kern(vmin_rows, d2t, itab)


def _k3_body(cd_ref, ci_ref, gid_ref):
    b = pl.program_id(0)
    d2 = cd_ref[0]
    idx = ci_ref[0].astype(jnp.float32)
    valid = d2 < jnp.float32(1e38)
    dist = jnp.sqrt(d2)
    rank = jnp.zeros((AB, CAP), jnp.int32)
    for j in range(CAP):
        dj = dist[:, j:j + 1]
        ij = idx[:, j:j + 1]
        vj = valid[:, j:j + 1]
        less = jnp.logical_and(
            vj, jnp.logical_or(dj < dist,
                               jnp.logical_and(dj == dist, ij < idx)))
        rank = rank + jnp.where(less, 1, 0)
    rank = jnp.where(valid, rank, jnp.int32(10000))
    base = (b // (A // AB)) * M
    gid = idx.astype(jnp.int32) + base
    cols = []
    for k in range(K_LOCAL):
        sel = rank == k
        cols.append(jnp.sum(jnp.where(sel, gid, 0), axis=-1, keepdims=True))
    gid_ref[0] = jnp.concatenate(cols, axis=-1)


def _run_k3(cd, ci):
    return pl.pallas_call(
        _k3_body,
        grid=(NQ // AB,),
        in_specs=[
            pl.BlockSpec((1, AB, CAP), lambda i: (i, 0, 0)),
            pl.BlockSpec((1, AB, CAP), lambda i: (i, 0, 0)),
        ],
        out_specs=pl.BlockSpec((1, AB, K_LOCAL), lambda i: (i, 0, 0)),
        out_shape=jax.ShapeDtypeStruct((NQ // AB, AB, K_LOCAL), jnp.int32),
    )(cd, ci)


def _k4_body(tab_hbm, gid_hbm, out_hbm, idx_v, rows_v, sem, semo):
    info = plsc.get_sparse_core_info()
    nc_sc = info.num_cores
    wid = lax.axis_index("s") * nc_sc + lax.axis_index("c")
    rpw = (NQ * K_LOCAL) // (nc_sc * 16)
    base = wid * rpw
    pltpu.sync_copy(gid_hbm.at[pl.ds(base, rpw)], idx_v)
    copies = []
    nfull = rpw // 128
    for c in range(nfull):
        copies.append(pltpu.async_copy(
            tab_hbm.at[idx_v.at[pl.ds(c * 128, 128)]],
            rows_v.at[pl.ds(c * 128, 128), :], sem))
    rem = rpw - nfull * 128
    if rem:
        copies.append(pltpu.async_copy(
            tab_hbm.at[idx_v.at[pl.ds(nfull * 128, rem)]],
            rows_v.at[pl.ds(nfull * 128, rem), :], sem))
    for cp in copies:
        cp.wait()
    oc = pltpu.async_copy(rows_v, out_hbm.at[pl.ds(base, rpw), :], semo)
    oc.wait()


def _run_k4(tab, gid_flat):
    mesh = plsc.VectorSubcoreMesh(core_axis_name="c", subcore_axis_name="s")
    rpw = (NQ * K_LOCAL) // 32
    kern = functools.partial(
        pl.kernel,
        mesh=mesh,
        compiler_params=pltpu.CompilerParams(use_tc_tiling_on_sc=False),
        out_type=jax.ShapeDtypeStruct((NQ * K_LOCAL, 8), jnp.float32),
        scratch_types=[
            pltpu.VMEM((rpw,), jnp.int32),
            pltpu.VMEM((rpw, 8), jnp.float32),
            pltpu.SemaphoreType.DMA,
            pltpu.SemaphoreType.DMA,
        ],
    )(_k4_body)
    return kern(tab, gid_flat)


def _mm(x3, W_ref, b_ref, kdim):
    acc = jnp.broadcast_to(b_ref[0:1, :][None], x3.shape[:2] + (EMB,))
    for k in range(kdim):
        acc = acc + x3[:, :, k:k + 1] * W_ref[k:k + 1, :][None]
    return acc


def _rel_part(feat, nfeat, axs, ays, mref, Wp_ref, bp_ref, We_ref, be_ref,
              kemb):
    N = feat.shape[0]
    fx = feat[:, 0:1][None]
    fy = feat[:, 1:2][None]
    px = fx - axs
    py = fy - ays
    pos = jnp.concatenate(
        [px, py,
         jnp.broadcast_to(feat[:, 2:3][None], (AB, N, 1)),
         jnp.broadcast_to(feat[:, 3:4][None], (AB, N, 1))], axis=-1)
    emb = _mm(pos, Wp_ref, bp_ref, 4)
    base = _mm(feat[None, :, :kemb], We_ref, be_ref, kemb)
    emb = emb + base
    dist = jnp.sqrt(px * px + py * py)
    msk = jnp.logical_or(mref[0, 0:1, :][:, :, None] > 0.5,
                         dist > 1.0)[:, :, 0]
    return emb, msk.astype(jnp.float32)


def _k5_body(m2a_ref, agt_ref, agf_ref, li_ref, st_ref, wa_ref,
             lm_ref, sm_ref, wm_ref, am_ref,
             wmap_ref, bmap_ref, wm2a_ref, bm2a_ref,
             wlight_ref, blight_ref, wl2a_ref, bl2a_ref,
             wstop_ref, bstop_ref, ws2a_ref, bs2a_ref,
             wwalk_ref, bwalk_ref, ww2a_ref, bw2a_ref,
             wego_ref, bego_ref, wag_ref, bag_ref, wa2a_ref, ba2a_ref,
             scene_o, a2a_o, aemb_o, smask_o, am_o):
    j = pl.program_id(1)
    agt = agt_ref[0]
    agf = agf_ref[0]
    axs = agt[:, 0:1][:, None, :]
    ays = agt[:, 1:2][:, None, :]

    f3 = m2a_ref[0].reshape(AB, K_LOCAL, 8)
    px = f3[:, :, 0:1] - axs
    py = f3[:, :, 1:2] - ays
    e0x = f3[:, :, 0:1] - agf[0, 0]
    e0y = f3[:, :, 1:2] - agf[0, 1]
    ego_d = jnp.sqrt(e0x * e0x + e0y * e0y)
    gmask = f3[:, :, 7:8]
    unobs = jnp.logical_or(ego_d > 1.0, gmask > 0.5)
    obs = jnp.where(unobs, 0.0, 1.0)
    pos4 = jnp.concatenate([px, py, f3[:, :, 2:3], f3[:, :, 3:4]], axis=-1)
    m2a = _mm(pos4 * obs, wm2a_ref, bm2a_ref, 4) + _mm(
        f3[:, :, :7], wmap_ref, bmap_ref, 7)
    mdist = jnp.sqrt(px * px + py * py)
    scene_o[0, :, 0:K_LOCAL, :] = m2a
    mmsk = jnp.logical_or(gmask > 0.5,
                          mdist > 1.0)[:, :, 0].astype(jnp.float32)

    l2a, lmsk = _rel_part(li_ref[0], 8, axs, ays, lm_ref, wl2a_ref, bl2a_ref,
                          wlight_ref, blight_ref, 8)
    scene_o[0, :, K_LOCAL:K_LOCAL + 16, :] = l2a
    s2a, smsk = _rel_part(st_ref[0], 5, axs, ays, sm_ref, ws2a_ref, bs2a_ref,
                          wstop_ref, bstop_ref, 5)
    scene_o[0, :, K_LOCAL + 16:K_LOCAL + 48, :] = s2a
    w2a, wmsk = _rel_part(wa_ref[0], 6, axs, ays, wm_ref, ww2a_ref, bw2a_ref,
                          wwalk_ref, bwalk_ref, 6)
    scene_o[0, :, K_LOCAL + 48:K_LOCAL + 112, :] = w2a
    smask_o[0] = jnp.concatenate([mmsk, lmsk, smsk, wmsk], axis=1)

    gx = agf[:, 0:1][None]
    gy = agf[:, 1:2][None]
    apx = gx - axs
    apy = gy - ays
    apos = jnp.concatenate(
        [apx, apy,
         jnp.broadcast_to(agf[:, 2:3][None], (AB, A, 1)),
         jnp.broadcast_to(agf[:, 3:4][None], (AB, A, 1))], axis=-1)
    a2a_o[0] = _mm(apos, wa2a_ref, ba2a_ref, 4)
    adist = jnp.sqrt(apx * apx + apy * apy)
    am_o[0] = jnp.logical_or(am_ref[0, 0:1, :][:, :, None] > 0.5,
                             adist > 1.0)[:, :, 0].astype(jnp.float32)

    ego = _mm(agt[None], wego_ref, bego_ref, 7)[0]
    oth = _mm(agt[None], wag_ref, bag_ref, 7)[0]
    rows = lax.broadcasted_iota(jnp.int32, (AB, 1), 0) + j * AB
    aemb_o[0] = jnp.where(rows == 0, ego, oth)


def _run_k5(m2a_rows, agents, lights, stops, walkers,
            lmask_f, smask_f, wmask_f, amask_f, weights):
    L, S, Wn = 16, 32, 64

    def wspec(w):
        return pl.BlockSpec(w.shape, lambda b, j: (0,) * w.ndim)

    win = []
    wargs = []
    for w in weights:
        w2 = w.reshape(1, EMB) if w.ndim == 1 else w
        wargs.append(w2)
        win.append(wspec(w2))

    def bo(shape):
        nd = len(shape)
        return pl.BlockSpec((1, 1) + shape,
                            lambda b, j: (b, j) + (0,) * nd)

    NS = K_LOCAL + L + S + Wn
    outs = [(NS, EMB), (A, EMB), (EMB,), (NS,), (A,)]
    return pl.pallas_call(
        _k5_body,
        grid=(B, A // AB),
        in_specs=[
            pl.BlockSpec((1, AB * K_LOCAL, 8),
                         lambda b, j: (b * (A // AB) + j, 0, 0)),
            pl.BlockSpec((1, AB, 7), lambda b, j: (b, j, 0)),
            pl.BlockSpec((1, A, 7), lambda b, j: (b, 0, 0)),
            pl.BlockSpec((1, L, 8), lambda b, j: (b, 0, 0)),
            pl.BlockSpec((1, S, 5), lambda b, j: (b, 0, 0)),
            pl.BlockSpec((1, Wn, 6), lambda b, j: (b, 0, 0)),
            pl.BlockSpec((1, 1, L), lambda b, j: (b, 0, 0)),
            pl.BlockSpec((1, 1, S), lambda b, j: (b, 0, 0)),
            pl.BlockSpec((1, 1, Wn), lambda b, j: (b, 0, 0)),
            pl.BlockSpec((1, 1, A), lambda b, j: (b, 0, 0)),
        ] + win,
        out_specs=[
            pl.BlockSpec((1, AB) + sh, (lambda b, j, n=len(sh): (b, j) + (0,) * n))
            for sh in outs
        ],
        out_shape=[
            jax.ShapeDtypeStruct((B, A) + sh, jnp.float32) for sh in outs
        ],
    )(m2a_rows.reshape(NQ // AB, AB * K_LOCAL, 8), agents, agents,
      lights, stops, walkers, lmask_f, smask_f, wmask_f, amask_f, *wargs)


def kernel(map_features, map_masks, light_features, light_masks, stop_features,
           stop_masks, walker_features, walker_masks, agents_features,
           agents_masks, W_map, b_map, W_m2a, b_m2a, W_light, b_light, W_l2a,
           b_l2a, W_stop, b_stop, W_s2a, b_s2a, W_walk, b_walk, W_w2a, b_w2a,
           W_ego, b_ego, W_ag, b_ag, W_a2a, b_a2a):
    mapt = jnp.transpose(map_features[:, :, :2], (0, 2, 1))
    mapt = jnp.pad(mapt, ((0, 0), (0, 0), (0, MP - M)))
    maskf = map_masks.astype(jnp.float32)[:, None, :]
    maskt = jnp.pad(maskf, ((0, 0), (0, 0), (0, MP - M)),
                    constant_values=1.0)
    tab = jnp.concatenate(
        [map_features, map_masks.astype(jnp.float32)[:, :, None]],
        axis=-1).reshape(B * M, 8)

    d2t, vmin = _run_k1(mapt, maskt, agents_features)
    vmin_rows = vmin.reshape(NQ, NVP)

    itab = jnp.arange(NC * CL, dtype=jnp.int32).reshape(NC, CL)
    cd, ci = _run_k2(vmin_rows, d2t, itab)
    cd3 = cd.reshape(NQ // AB, AB, CAP)
    ci3 = ci.reshape(NQ // AB, AB, CAP)

    gid = _run_k3(cd3, ci3)
    gid_flat = gid.reshape(NQ * K_LOCAL)

    rows = _run_k4(tab, gid_flat)
    m2a_rows = rows

    weights = (W_map, b_map, W_m2a, b_m2a, W_light, b_light, W_l2a, b_l2a,
               W_stop, b_stop, W_s2a, b_s2a, W_walk, b_walk, W_w2a, b_w2a,
               W_ego, b_ego, W_ag, b_ag, W_a2a, b_a2a)
    (scene_emb, a2a_emb, agents_emb, smask_f, am_f) = _run_k5(
        m2a_rows, agents_features, light_features, stop_features,
        walker_features,
        light_masks.astype(jnp.float32)[:, None, :],
        stop_masks.astype(jnp.float32)[:, None, :],
        walker_masks.astype(jnp.float32)[:, None, :],
        agents_masks.astype(jnp.float32)[:, None, :],
        weights)
    return (scene_emb, smask_f > 0.5, agents_emb, a2a_emb, am_f > 0.5)


# speedup vs baseline: 9.7264x; 2.1134x over previous
import functools

import jax
import jax.numpy as jnp
from jax import lax
from jax.experimental import pallas as pl
from jax.experimental.pallas import tpu as pltpu
from jax.experimental.pallas import tpu_sc as plsc

EMB = 128
K_LOCAL = 50
B, M, A = 16, 20000, 64
CL = 128
NC = 160
MP = NC * CL
CAP = 128
AB = 8
NQ = B * A
INF = float("inf")


NVP = NC + 16
NSEL = 64


def _k1_body(mapt_ref, maskt_ref, ag_ref, d2_ref, vmin_ref):
    qx = ag_ref[0, :, 0:1]
    qy = ag_ref[0, :, 1:2]
    mx = mapt_ref[0, 0:1, :]
    my = mapt_ref[0, 1:2, :]
    mk = maskt_ref[0, 0:1, :]
    dx = mx - qx
    dy = my - qy
    d2 = dx * dx + dy * dy
    d2 = jnp.where(mk > 0.5, INF, d2)
    d2r = d2.reshape(AB * NC, CL)
    d2_ref[...] = d2r
    vmin_ref[0, 0, :] = jnp.min(d2r, axis=-1)


def _run_k1(mapt, maskt, agents):
    return pl.pallas_call(
        _k1_body,
        grid=(B, A // AB),
        in_specs=[
            pl.BlockSpec((1, 2, MP), lambda b, j: (b, 0, 0)),
            pl.BlockSpec((1, 1, MP), lambda b, j: (b, 0, 0)),
            pl.BlockSpec((1, AB, 7), lambda b, j: (b, j, 0)),
        ],
        out_specs=[
            pl.BlockSpec((AB * NC, CL),
                         lambda b, j: (b * (A // AB) + j, 0)),
            pl.BlockSpec((1, 1, AB * NC),
                         lambda b, j: (b * (A // AB) + j, 0, 0)),
        ],
        out_shape=[
            jax.ShapeDtypeStruct((NQ * NC, CL), jnp.float32),
            jax.ShapeDtypeStruct((NQ // AB, 1, AB * NC), jnp.float32),
        ],
    )(mapt, maskt, agents)


def _k1b_body(cm_ref, vmin_ref):
    vm = cm_ref[...]

    def step(_, t):
        masked = jnp.where(vm > t, vm, INF)
        return jnp.min(masked, axis=-1, keepdims=True)

    t = lax.fori_loop(0, K_LOCAL, step,
                      jnp.full((vm.shape[0], 1), -INF, jnp.float32))
    vmin_ref[:, 0:NC] = vm
    vmin_ref[:, NC:NVP] = jnp.broadcast_to(t, (vm.shape[0], 16))


def _run_k1b(cmin):
    QB = 256
    return pl.pallas_call(
        _k1b_body,
        grid=(NQ // QB,),
        in_specs=[pl.BlockSpec((QB, NC), lambda i: (i, 0))],
        out_specs=pl.BlockSpec((QB, NVP), lambda i: (i, 0)),
        out_shape=jax.ShapeDtypeStruct((NQ, NVP), jnp.float32),
    )(cmin)


def _gather16(x, idx):
    return x.at[idx].get(mode="promise_in_bounds")


def _prefix16(msk, lane):
    rank = jnp.where(msk, 1, 0).astype(jnp.int32)
    for k in (1, 2, 4, 8):
        sh = _gather16(rank, jnp.maximum(lane - k, 0))
        rank = rank + jnp.where(lane >= k, sh, 0)
    return rank


def _compact16(vals, rank, lane, pad):
    tgt = lane + 1
    p = jnp.where(rank[7] < tgt, 8, 0).astype(jnp.int32)
    for step in (4, 2, 1):
        r = _gather16(rank, p + (step - 1))
        p = p + jnp.where(r < tgt, step, 0)
    okm = tgt <= rank[15]
    out0 = jnp.where(okm, _gather16(vals[0], p), pad)
    return [out0] + [_gather16(v, p) for v in vals[1:]]


def _k2_body(vmin_hbm, d2t_hbm, itab_hbm, cd_hbm, ci_hbm,
             vrow0, vrow1, idsl_v, idsg_v, gd2_v, gidx_v, cd_v, ci_v,
             sem0, sem1, semg, semo):
    info = plsc.get_sparse_core_info()
    nc_sc = info.num_cores
    wid = lax.axis_index("s") * nc_sc + lax.axis_index("c")
    qpw = NQ // (nc_sc * 16)
    qbase = wid * qpw
    lane = lax.iota(jnp.int32, 16)
    l15 = jnp.full((16,), 15, jnp.int32)
    vrows = (vrow0, vrow1)
    sems = (sem0, sem1)
    cp_cur = pltpu.async_copy(vmin_hbm.at[qbase], vrows[0], sems[0])

    for q in range(qpw):
        cp_next = None
        if q + 1 < qpw:
            cp_next = pltpu.async_copy(vmin_hbm.at[qbase + q + 1],
                                       vrows[(q + 1) % 2], sems[(q + 1) % 2])
        cp_cur.wait()
        vrow = vrows[q % 2]
        tau16 = vrow[pl.ds(NC, 16)]

        for j in range(NSEL // 16):
            idsl_v[pl.ds(j * 16, 16)] = jnp.full((16,), NC - 1, jnp.int32)

        def stepA(j, off):
            vm = vrow[pl.ds(j * 16, 16)]
            msk = vm <= tau16
            rank = _prefix16(msk, lane)
            n = rank[15]

            @pl.when(n > 0)
            def _():
                (ids,) = _compact16([lane + j * 16], rank, lane, NC - 1)
                offc = jnp.minimum(off, NSEL)
                idsl_v[pl.ds(offc, 16)] = ids

            return off + n

        na = lax.fori_loop(0, NC // 16, stepA, jnp.int32(0))
        na = jnp.minimum(na, NSEL)

        gbase = (qbase + q) * NC
        for j in range(NSEL // 16):
            idsg_v[pl.ds(j * 16, 16)] = idsl_v[pl.ds(j * 16, 16)] + gbase
        gc0 = pltpu.async_copy(d2t_hbm.at[idsg_v], gd2_v, semg)
        gc1 = pltpu.async_copy(itab_hbm.at[idsl_v.at[pl.ds(0, NSEL)]],
                               gidx_v, semg)
        gc0.wait()
        gc1.wait()
        for j in range(CAP // 16 + 1):
            cd_v[pl.ds(j * 16, 16)] = jnp.full((16,), INF, jnp.float32)
            ci_v[pl.ds(j * 16, 16)] = jnp.zeros((16,), jnp.int32)

        def stepB(j, off):
            row = gd2_v[j // 8, pl.ds((j % 8) * 16, 16)]
            msk = row <= tau16
            rank = _prefix16(msk, lane)
            n = rank[15]

            @pl.when(n > 0)
            def _():
                idxv = gidx_v[j // 8, pl.ds((j % 8) * 16, 16)]
                crow, cidx = _compact16([row, idxv], rank, lane, INF)
                offc = jnp.minimum(off, CAP)
                cd_v[pl.ds(offc, 16)] = crow
                ci_v[pl.ds(offc, 16)] = cidx

            return off + n

        lax.fori_loop(0, na * 8, stepB, jnp.int32(0))
        oc0 = pltpu.async_copy(cd_v.at[pl.ds(0, CAP)], cd_hbm.at[qbase + q],
                               semo)
        oc1 = pltpu.async_copy(ci_v.at[pl.ds(0, CAP)], ci_hbm.at[qbase + q],
                               semo)
        oc0.wait()
        oc1.wait()
        cp_cur = cp_next


def _run_k2(vmin_rows, d2t, itab):
    mesh = plsc.VectorSubcoreMesh(core_axis_name="c", subcore_axis_name="s")
    kern = functools.partial(
        pl.kernel,
        mesh=mesh,
        out_type=[
            jax.ShapeDtypeStruct((NQ, CAP), jnp.float32),
            jax.ShapeDtypeStruct((NQ, CAP), jnp.int32),
        ],
        scratch_types=[
            pltpu.VMEM((NVP,), jnp.float32),
            pltpu.VMEM((NVP,), jnp.float32),
            pltpu.VMEM((128,), jnp.int32),
            pltpu.VMEM((NSEL,), jnp.int32),
            pltpu.VMEM((NSEL, CL), jnp.float32),
            pltpu.VMEM((NSEL, CL), jnp.int32),
            pltpu.VMEM((CAP + 128,), jnp.float32),
            pltpu.VMEM((CAP + 128,), jnp.int32),
            pltpu.SemaphoreType.DMA,
            pltpu.SemaphoreType.DMA,
            pltpu.SemaphoreType.DMA,
            pltpu.SemaphoreType.DMA,
        ],
    )(_k2_body)
    return kern(vmin_rows, d2t, itab)


def _k3_body(cd_ref, ci_ref, gid_ref):
    b = pl.program_id(0)
    d2 = cd_ref[0]
    idx = ci_ref[0].astype(jnp.float32)
    valid = d2 < jnp.float32(1e38)
    dist = jnp.sqrt(d2)
    rank = jnp.zeros((AB, CAP), jnp.int32)
    for j in range(CAP):
        dj = dist[:, j:j + 1]
        ij = idx[:, j:j + 1]
        vj = valid[:, j:j + 1]
        less = jnp.logical_and(
            vj, jnp.logical_or(dj < dist,
                               jnp.logical_and(dj == dist, ij < idx)))
        rank = rank + jnp.where(less, 1, 0)
    rank = jnp.where(valid, rank, jnp.int32(10000))
    base = (b // (A // AB)) * M
    gid = idx.astype(jnp.int32) + base
    cols = []
    for k in range(K_LOCAL):
        sel = rank == k
        cols.append(jnp.sum(jnp.where(sel, gid, 0), axis=-1, keepdims=True))
    gid_ref[0] = jnp.concatenate(cols, axis=-1)


def _run_k3(cd, ci):
    return pl.pallas_call(
        _k3_body,
        grid=(NQ // AB,),
        in_specs=[
            pl.BlockSpec((1, AB, CAP), lambda i: (i, 0, 0)),
            pl.BlockSpec((1, AB, CAP), lambda i: (i, 0, 0)),
        ],
        out_specs=pl.BlockSpec((1, AB, K_LOCAL), lambda i: (i, 0, 0)),
        out_shape=jax.ShapeDtypeStruct((NQ // AB, AB, K_LOCAL), jnp.int32),
    )(cd, ci)


def _k4_body(tab_hbm, gid_hbm, out_hbm, idx_v, rows_v, sem, semo):
    info = plsc.get_sparse_core_info()
    nc_sc = info.num_cores
    wid = lax.axis_index("s") * nc_sc + lax.axis_index("c")
    rpw = (NQ * K_LOCAL) // (nc_sc * 16)
    base = wid * rpw
    pltpu.sync_copy(gid_hbm.at[pl.ds(base, rpw)], idx_v)
    copies = []
    nfull = rpw // 128
    for c in range(nfull):
        copies.append(pltpu.async_copy(
            tab_hbm.at[idx_v.at[pl.ds(c * 128, 128)]],
            rows_v.at[pl.ds(c * 128, 128), :], sem))
    rem = rpw - nfull * 128
    if rem:
        copies.append(pltpu.async_copy(
            tab_hbm.at[idx_v.at[pl.ds(nfull * 128, rem)]],
            rows_v.at[pl.ds(nfull * 128, rem), :], sem))
    for cp in copies:
        cp.wait()
    oc = pltpu.async_copy(rows_v, out_hbm.at[pl.ds(base, rpw), :], semo)
    oc.wait()


def _run_k4(tab, gid_flat):
    mesh = plsc.VectorSubcoreMesh(core_axis_name="c", subcore_axis_name="s")
    rpw = (NQ * K_LOCAL) // 32
    kern = functools.partial(
        pl.kernel,
        mesh=mesh,
        compiler_params=pltpu.CompilerParams(use_tc_tiling_on_sc=False),
        out_type=jax.ShapeDtypeStruct((NQ * K_LOCAL, 8), jnp.float32),
        scratch_types=[
            pltpu.VMEM((rpw,), jnp.int32),
            pltpu.VMEM((rpw, 8), jnp.float32),
            pltpu.SemaphoreType.DMA,
            pltpu.SemaphoreType.DMA,
        ],
    )(_k4_body)
    return kern(tab, gid_flat)


def _mm(x3, W_ref, b_ref, kdim):
    acc = jnp.broadcast_to(b_ref[0:1, :][None], x3.shape[:2] + (EMB,))
    for k in range(kdim):
        acc = acc + x3[:, :, k:k + 1] * W_ref[k:k + 1, :][None]
    return acc


def _rel_part(feat, nfeat, axs, ays, mref, Wp_ref, bp_ref, We_ref, be_ref,
              kemb):
    N = feat.shape[0]
    fx = feat[:, 0:1][None]
    fy = feat[:, 1:2][None]
    px = fx - axs
    py = fy - ays
    pos = jnp.concatenate(
        [px, py,
         jnp.broadcast_to(feat[:, 2:3][None], (AB, N, 1)),
         jnp.broadcast_to(feat[:, 3:4][None], (AB, N, 1))], axis=-1)
    emb = _mm(pos, Wp_ref, bp_ref, 4)
    base = _mm(feat[None, :, :kemb], We_ref, be_ref, kemb)
    emb = emb + base
    dist = jnp.sqrt(px * px + py * py)
    msk = jnp.logical_or(mref[0, 0:1, :][:, :, None] > 0.5,
                         dist > 1.0)[:, :, 0]
    return emb, msk.astype(jnp.float32)


def _k5_body(m2a_ref, agt_ref, agf_ref, li_ref, st_ref, wa_ref,
             lm_ref, sm_ref, wm_ref, am_ref,
             wmap_ref, bmap_ref, wm2a_ref, bm2a_ref,
             wlight_ref, blight_ref, wl2a_ref, bl2a_ref,
             wstop_ref, bstop_ref, ws2a_ref, bs2a_ref,
             wwalk_ref, bwalk_ref, ww2a_ref, bw2a_ref,
             wego_ref, bego_ref, wag_ref, bag_ref, wa2a_ref, ba2a_ref,
             scene_o, a2a_o, aemb_o, smask_o, am_o):
    j = pl.program_id(1)
    agt = agt_ref[0]
    agf = agf_ref[0]
    axs = agt[:, 0:1][:, None, :]
    ays = agt[:, 1:2][:, None, :]

    f3 = m2a_ref[0].reshape(AB, K_LOCAL, 8)
    px = f3[:, :, 0:1] - axs
    py = f3[:, :, 1:2] - ays
    e0x = f3[:, :, 0:1] - agf[0, 0]
    e0y = f3[:, :, 1:2] - agf[0, 1]
    ego_d = jnp.sqrt(e0x * e0x + e0y * e0y)
    gmask = f3[:, :, 7:8]
    unobs = jnp.logical_or(ego_d > 1.0, gmask > 0.5)
    obs = jnp.where(unobs, 0.0, 1.0)
    pos4 = jnp.concatenate([px, py, f3[:, :, 2:3], f3[:, :, 3:4]], axis=-1)
    m2a = _mm(pos4 * obs, wm2a_ref, bm2a_ref, 4) + _mm(
        f3[:, :, :7], wmap_ref, bmap_ref, 7)
    mdist = jnp.sqrt(px * px + py * py)
    scene_o[0, :, 0:K_LOCAL, :] = m2a
    mmsk = jnp.logical_or(gmask > 0.5,
                          mdist > 1.0)[:, :, 0].astype(jnp.float32)

    l2a, lmsk = _rel_part(li_ref[0], 8, axs, ays, lm_ref, wl2a_ref, bl2a_ref,
                          wlight_ref, blight_ref, 8)
    scene_o[0, :, K_LOCAL:K_LOCAL + 16, :] = l2a
    s2a, smsk = _rel_part(st_ref[0], 5, axs, ays, sm_ref, ws2a_ref, bs2a_ref,
                          wstop_ref, bstop_ref, 5)
    scene_o[0, :, K_LOCAL + 16:K_LOCAL + 48, :] = s2a
    w2a, wmsk = _rel_part(wa_ref[0], 6, axs, ays, wm_ref, ww2a_ref, bw2a_ref,
                          wwalk_ref, bwalk_ref, 6)
    scene_o[0, :, K_LOCAL + 48:K_LOCAL + 112, :] = w2a
    smask_o[0] = jnp.concatenate([mmsk, lmsk, smsk, wmsk], axis=1)

    gx = agf[:, 0:1][None]
    gy = agf[:, 1:2][None]
    apx = gx - axs
    apy = gy - ays
    apos = jnp.concatenate(
        [apx, apy,
         jnp.broadcast_to(agf[:, 2:3][None], (AB, A, 1)),
         jnp.broadcast_to(agf[:, 3:4][None], (AB, A, 1))], axis=-1)
    a2a_o[0] = _mm(apos, wa2a_ref, ba2a_ref, 4)
    adist = jnp.sqrt(apx * apx + apy * apy)
    am_o[0] = jnp.logical_or(am_ref[0, 0:1, :][:, :, None] > 0.5,
                             adist > 1.0)[:, :, 0].astype(jnp.float32)

    ego = _mm(agt[None], wego_ref, bego_ref, 7)[0]
    oth = _mm(agt[None], wag_ref, bag_ref, 7)[0]
    rows = lax.broadcasted_iota(jnp.int32, (AB, 1), 0) + j * AB
    aemb_o[0] = jnp.where(rows == 0, ego, oth)


def _run_k5(m2a_rows, agents, lights, stops, walkers,
            lmask_f, smask_f, wmask_f, amask_f, weights):
    L, S, Wn = 16, 32, 64

    def wspec(w):
        return pl.BlockSpec(w.shape, lambda b, j: (0,) * w.ndim)

    win = []
    wargs = []
    for w in weights:
        w2 = w.reshape(1, EMB) if w.ndim == 1 else w
        wargs.append(w2)
        win.append(wspec(w2))

    def bo(shape):
        nd = len(shape)
        return pl.BlockSpec((1, 1) + shape,
                            lambda b, j: (b, j) + (0,) * nd)

    NS = K_LOCAL + L + S + Wn
    outs = [(NS, EMB), (A, EMB), (EMB,), (NS,), (A,)]
    return pl.pallas_call(
        _k5_body,
        grid=(B, A // AB),
        in_specs=[
            pl.BlockSpec((1, AB * K_LOCAL, 8),
                         lambda b, j: (b * (A // AB) + j, 0, 0)),
            pl.BlockSpec((1, AB, 7), lambda b, j: (b, j, 0)),
            pl.BlockSpec((1, A, 7), lambda b, j: (b, 0, 0)),
            pl.BlockSpec((1, L, 8), lambda b, j: (b, 0, 0)),
            pl.BlockSpec((1, S, 5), lambda b, j: (b, 0, 0)),
            pl.BlockSpec((1, Wn, 6), lambda b, j: (b, 0, 0)),
            pl.BlockSpec((1, 1, L), lambda b, j: (b, 0, 0)),
            pl.BlockSpec((1, 1, S), lambda b, j: (b, 0, 0)),
            pl.BlockSpec((1, 1, Wn), lambda b, j: (b, 0, 0)),
            pl.BlockSpec((1, 1, A), lambda b, j: (b, 0, 0)),
        ] + win,
        out_specs=[
            pl.BlockSpec((1, AB) + sh, (lambda b, j, n=len(sh): (b, j) + (0,) * n))
            for sh in outs
        ],
        out_shape=[
            jax.ShapeDtypeStruct((B, A) + sh, jnp.float32) for sh in outs
        ],
    )(m2a_rows.reshape(NQ // AB, AB * K_LOCAL, 8), agents, agents,
      lights, stops, walkers, lmask_f, smask_f, wmask_f, amask_f, *wargs)


def kernel(map_features, map_masks, light_features, light_masks, stop_features,
           stop_masks, walker_features, walker_masks, agents_features,
           agents_masks, W_map, b_map, W_m2a, b_m2a, W_light, b_light, W_l2a,
           b_l2a, W_stop, b_stop, W_s2a, b_s2a, W_walk, b_walk, W_w2a, b_w2a,
           W_ego, b_ego, W_ag, b_ag, W_a2a, b_a2a):
    mapt = jnp.transpose(map_features[:, :, :2], (0, 2, 1))
    mapt = jnp.pad(mapt, ((0, 0), (0, 0), (0, MP - M)))
    maskf = map_masks.astype(jnp.float32)[:, None, :]
    maskt = jnp.pad(maskf, ((0, 0), (0, 0), (0, MP - M)),
                    constant_values=1.0)
    tab = jnp.concatenate(
        [map_features, map_masks.astype(jnp.float32)[:, :, None]],
        axis=-1).reshape(B * M, 8)

    d2t, cmin = _run_k1(mapt, maskt, agents_features)
    vmin_rows = _run_k1b(cmin.reshape(NQ, NC))

    itab = jnp.arange(NC * CL, dtype=jnp.int32).reshape(NC, CL)
    cd, ci = _run_k2(vmin_rows, d2t, itab)
    cd3 = cd.reshape(NQ // AB, AB, CAP)
    ci3 = ci.reshape(NQ // AB, AB, CAP)

    gid = _run_k3(cd3, ci3)
    gid_flat = gid.reshape(NQ * K_LOCAL)

    rows = _run_k4(tab, gid_flat)
    m2a_rows = rows

    weights = (W_map, b_map, W_m2a, b_m2a, W_light, b_light, W_l2a, b_l2a,
               W_stop, b_stop, W_s2a, b_s2a, W_walk, b_walk, W_w2a, b_w2a,
               W_ego, b_ego, W_ag, b_ag, W_a2a, b_a2a)
    (scene_emb, a2a_emb, agents_emb, smask_f, am_f) = _run_k5(
        m2a_rows, agents_features, light_features, stop_features,
        walker_features,
        light_masks.astype(jnp.float32)[:, None, :],
        stop_masks.astype(jnp.float32)[:, None, :],
        walker_masks.astype(jnp.float32)[:, None, :],
        agents_masks.astype(jnp.float32)[:, None, :],
        weights)
    return (scene_emb, smask_f > 0.5, agents_emb, a2a_emb, am_f > 0.5)
